# Initial kernel scaffold; baseline (speedup 1.0000x reference)
import functools

import jax
import jax.numpy as jnp
from jax import lax
from jax.experimental import pallas as pl
from jax.experimental.pallas import tpu as pltpu
from jax.experimental.pallas import tpu_sc as plsc

NC = 2
NS = 16
NW = NC * NS
LANES = 16
CHUNK = 128
HR = 640


def _row_split(n_rows):
    per = ((n_rows + NS - 1) // NS + 7) // 8 * 8
    last = n_rows - per * (NS - 1)
    assert last > 0
    return per, last


def _sc_degrees(src_flat, dst_flat, iota_rows):
    e = src_flat.shape[0]
    n_chunks = e // CHUNK
    per_tile = (n_chunks + NW - 1) // NW
    mesh = plsc.VectorSubcoreMesh(core_axis_name="c", subcore_axis_name="s")

    @functools.partial(
        pl.kernel,
        out_type=jax.ShapeDtypeStruct((NC, 2, HR, LANES), jnp.float32),
        mesh=mesh,
        scratch_types=[
            pltpu.VMEM((CHUNK,), jnp.int32),
            pltpu.VMEM((CHUNK,), jnp.int32),
            pltpu.VMEM((HR, LANES), jnp.float32),
            pltpu.VMEM((HR, LANES), jnp.float32),
            pltpu.VMEM((HR // CHUNK, CHUNK), jnp.int32),
            pltpu.VMEM_SHARED((HR, LANES), jnp.float32),
            pltpu.VMEM_SHARED((HR, LANES), jnp.float32),
        ],
    )
    def k(src_h, dst_h, iota_h, out_h, sbuf, dbuf, sh_v, dh_v, iid_v, ssh, dsh):
        cid = lax.axis_index("c")
        sid = lax.axis_index("s")
        wid = sid * NC + cid
        z116 = jnp.zeros((1, LANES), jnp.float32)
        ones16 = jnp.ones((LANES,), jnp.float32)

        @pl.loop(0, HR)
        def _(r):
            sh_v[pl.ds(r, 1), :] = z116
            dh_v[pl.ds(r, 1), :] = z116

        rows_per = HR // NS
        r0 = sid * rows_per
        pltpu.sync_copy(sh_v.at[pl.ds(r0, rows_per)], ssh.at[pl.ds(r0, rows_per)])
        pltpu.sync_copy(dh_v.at[pl.ds(r0, rows_per)], dsh.at[pl.ds(r0, rows_per)])
        pltpu.sync_copy(iota_h, iid_v)
        plsc.subcore_barrier()

        @pl.loop(0, per_tile)
        def _(ci):
            g = ci * NW + wid

            @pl.when(g < n_chunks)
            def _():
                pltpu.sync_copy(src_h.at[pl.ds(g * CHUNK, CHUNK)], sbuf)
                pltpu.sync_copy(dst_h.at[pl.ds(g * CHUNK, CHUNK)], dbuf)
                for j in range(CHUNK // LANES):
                    sv = sbuf[pl.ds(j * LANES, LANES)]
                    plsc.addupdate_scatter(
                        sh_v,
                        [lax.shift_right_logical(sv, 4), lax.bitwise_and(sv, 15)],
                        ones16,
                    )
                    dv = dbuf[pl.ds(j * LANES, LANES)]
                    plsc.addupdate_scatter(
                        dh_v,
                        [lax.shift_right_logical(dv, 4), lax.bitwise_and(dv, 15)],
                        ones16,
                    )

        for r in range(HR // CHUNK):
            pltpu.sync_copy(sh_v.at[pl.ds(r * CHUNK, CHUNK)],
                            ssh.at[iid_v.at[r]], add=True)
            pltpu.sync_copy(dh_v.at[pl.ds(r * CHUNK, CHUNK)],
                            dsh.at[iid_v.at[r]], add=True)
        plsc.subcore_barrier()

        pltpu.sync_copy(ssh.at[pl.ds(r0, rows_per)],
                        out_h.at[cid, 0, pl.ds(r0, rows_per)])
        pltpu.sync_copy(dsh.at[pl.ds(r0, rows_per)],
                        out_h.at[cid, 1, pl.ds(r0, rows_per)])

    return k(src_flat, dst_flat, iota_rows)


def _sc_scatter(h2, dst2d, src_flat):
    n, d = h2.shape
    n_chunks = dst2d.shape[0]
    per_tile = (n_chunks + NW - 1) // NW
    rows_per, rows_last = _row_split(n)
    mesh = plsc.VectorSubcoreMesh(core_axis_name="c", subcore_axis_name="s")

    @functools.partial(
        pl.kernel,
        out_type=jax.ShapeDtypeStruct((NC, n, d), jnp.float32),
        mesh=mesh,
        scratch_types=[
            pltpu.VMEM((CHUNK,), jnp.int32),
            pltpu.VMEM((1, CHUNK), jnp.int32),
            pltpu.VMEM((CHUNK, d), jnp.float32),
            pltpu.VMEM((16, d), jnp.float32),
            pltpu.VMEM_SHARED((n, d), jnp.float32),
            pltpu.SemaphoreType.DMA,
        ],
    )
    def k(h2_h, d_h, s_h, out_h, sidx, didx, rows, zbuf, pooled, sem):
        cid = lax.axis_index("c")
        sid = lax.axis_index("s")
        wid = sid * NC + cid
        z116 = jnp.zeros((1, LANES), jnp.float32)

        for i in range(16):
            for j in range(d // LANES):
                zbuf[pl.ds(i, 1), pl.ds(j * LANES, LANES)] = z116

        r0 = sid * rows_per

        @pl.when(sid < NS - 1)
        def _():
            @pl.loop(0, rows_per // 16)
            def _(t):
                pltpu.sync_copy(zbuf, pooled.at[pl.ds(r0 + t * 16, 16)])

        @pl.when(sid == NS - 1)
        def _():
            @pl.loop(0, rows_last // 16)
            def _(t):
                pltpu.sync_copy(zbuf, pooled.at[pl.ds(r0 + t * 16, 16)])

        plsc.subcore_barrier()

        @pl.loop(0, per_tile)
        def _(ci):
            g = ci * NW + wid

            @pl.when(g < n_chunks)
            def _():
                pltpu.sync_copy(s_h.at[pl.ds(g * CHUNK, CHUNK)], sidx)
                pltpu.sync_copy(d_h.at[pl.ds(g, 1)], didx)
                pltpu.async_copy(h2_h.at[sidx], rows, sem).wait()
                pltpu.sync_copy(rows, pooled.at[didx.at[0]], add=True)

        plsc.subcore_barrier()

        @pl.when(sid < NS - 1)
        def _():
            pltpu.sync_copy(pooled.at[pl.ds(r0, rows_per)],
                            out_h.at[cid, pl.ds(r0, rows_per)])

        @pl.when(sid == NS - 1)
        def _():
            pltpu.sync_copy(pooled.at[pl.ds(r0, rows_last)],
                            out_h.at[cid, pl.ds(r0, rows_last)])

    return k(h2, dst2d, src_flat)


def _dot(a, b):
    return lax.dot_general(a, b, (((1,), (0,)), ((), ())),
                           precision=lax.Precision.HIGHEST,
                           preferred_element_type=jnp.float32)


def _mm_relu_body(x_ref, w_ref, b_ref, o_ref):
    o_ref[...] = jnp.maximum(_dot(x_ref[...], w_ref[...]) + b_ref[...], 0.0)


def _tc_mm_relu(x, w, b_row, block_rows):
    n, d = x.shape
    h = w.shape[1]
    grid = (n // block_rows,)
    return pl.pallas_call(
        _mm_relu_body,
        grid=grid,
        in_specs=[
            pl.BlockSpec((block_rows, d), lambda i: (i, 0)),
            pl.BlockSpec((d, h), lambda i: (0, 0)),
            pl.BlockSpec((1, h), lambda i: (0, 0)),
        ],
        out_specs=pl.BlockSpec((block_rows, h), lambda i: (i, 0)),
        out_shape=jax.ShapeDtypeStruct((n, h), jnp.float32),
    )(x, w, b_row)


def _scale_body(hist_ref, h_ref, h2_ref, rs_ref):
    out_deg = hist_ref[0, 0, :] + hist_ref[1, 0, :]
    in_deg = hist_ref[0, 1, :] + hist_ref[1, 1, :]
    rs_out = lax.rsqrt(jnp.maximum(out_deg, 1.0))
    h2_ref[...] = h_ref[...] * rs_out[:, None]
    rs_ref[...] = lax.rsqrt(jnp.maximum(in_deg, 1.0))[:, None]


def _tc_scale(hist_n, h_relu, block_rows):
    n, d = h_relu.shape
    grid = (n // block_rows,)
    return pl.pallas_call(
        _scale_body,
        grid=grid,
        in_specs=[
            pl.BlockSpec((NC, 2, block_rows), lambda i: (0, 0, i)),
            pl.BlockSpec((block_rows, d), lambda i: (i, 0)),
        ],
        out_specs=[
            pl.BlockSpec((block_rows, d), lambda i: (i, 0)),
            pl.BlockSpec((block_rows, 1), lambda i: (i, 0)),
        ],
        out_shape=[
            jax.ShapeDtypeStruct((n, d), jnp.float32),
            jax.ShapeDtypeStruct((n, 1), jnp.float32),
        ],
    )(hist_n, h_relu)


def _final_body(p_ref, rs_ref, w_ref, b_ref, x_ref, o_ref):
    s = (p_ref[0] + p_ref[1]) * rs_ref[...]
    o_ref[...] = jnp.maximum(_dot(s, w_ref[...]) + b_ref[...], 0.0) + x_ref[...]


def _tc_final(pooled2, rs_col, w, b_row, x, block_rows):
    n, d = x.shape
    h = w.shape[1]
    grid = (n // block_rows,)
    return pl.pallas_call(
        _final_body,
        grid=grid,
        in_specs=[
            pl.BlockSpec((NC, block_rows, h), lambda i: (0, i, 0)),
            pl.BlockSpec((block_rows, 1), lambda i: (i, 0)),
            pl.BlockSpec((h, h), lambda i: (0, 0)),
            pl.BlockSpec((1, h), lambda i: (0, 0)),
            pl.BlockSpec((block_rows, d), lambda i: (i, 0)),
        ],
        out_specs=pl.BlockSpec((block_rows, h), lambda i: (i, 0)),
        out_shape=jax.ShapeDtypeStruct((n, h), jnp.float32),
    )(pooled2, rs_col, w, b_row, x)


def kernel(x, edge_index, W_edge, b_edge, W_node, b_node):
    x = x.astype(jnp.float32)
    ei = edge_index.astype(jnp.int32)
    src = ei[0]
    dst = ei[1]
    e = src.shape[0]
    n = x.shape[0]
    dst2d = dst.reshape(e // CHUNK, CHUNK)
    iota_rows = jnp.arange(HR, dtype=jnp.int32).reshape(HR // CHUNK, CHUNK)

    block_rows = 2000

    hist = _sc_degrees(src, dst, iota_rows)
    h_relu = _tc_mm_relu(x, W_edge, b_edge.reshape(1, -1), block_rows)
    hist_n = hist.reshape(NC, 2, HR * LANES)[:, :, :n]
    h2, rs_col = _tc_scale(hist_n, h_relu, block_rows)
    pooled2 = _sc_scatter(h2, dst2d, src)
    out = _tc_final(pooled2, rs_col, W_node, b_node.reshape(1, -1), x,
                    block_rows)
    return out


# speedup vs baseline: 9.5566x; 9.5566x over previous
import dataclasses
import functools

import jax
import jax.numpy as jnp
from jax import lax
from jax.experimental import pallas as pl
from jax.experimental.pallas import tpu as pltpu
from jax.experimental.pallas import tpu_sc as plsc

NC = 2
NS = 16
NW = NC * NS
LANES = 16
CHUNK = 128
HROWS = 80


def _sc_compiler_params():
    cp = pltpu.CompilerParams()
    if "needs_layout_passes" in pltpu.CompilerParams.__dataclass_fields__:
        cp = dataclasses.replace(cp, needs_layout_passes=False)
    return cp


def _sc_degrees(src_flat, dst_flat, iota_rows):
    e = src_flat.shape[0]
    n_chunks = e // CHUNK
    per_tile = (n_chunks + NW - 1) // NW
    mesh = plsc.VectorSubcoreMesh(core_axis_name="c", subcore_axis_name="s")

    @functools.partial(
        pl.kernel,
        out_type=jax.ShapeDtypeStruct((NC, 2, HROWS, 128), jnp.float32),
        mesh=mesh,
        scratch_types=[
            pltpu.VMEM((CHUNK,), jnp.int32),
            pltpu.VMEM((CHUNK,), jnp.int32),
            pltpu.VMEM((HROWS, 128), jnp.float32),
            pltpu.VMEM((HROWS, 128), jnp.float32),
            pltpu.VMEM((1, HROWS), jnp.int32),
            pltpu.VMEM_SHARED((HROWS, 128), jnp.float32),
            pltpu.VMEM_SHARED((HROWS, 128), jnp.float32),
        ],
        compiler_params=_sc_compiler_params(),
    )
    def k(src_h, dst_h, iota_h, out_h, sbuf, dbuf, sh_v, dh_v, iid_v, ssh, dsh):
        cid = lax.axis_index("c")
        sid = lax.axis_index("s")
        wid = sid * NC + cid
        z16 = jnp.zeros((LANES,), jnp.float32)
        ones16 = jnp.ones((LANES,), jnp.float32)

        @pl.loop(0, HROWS)
        def _(r):
            for j in range(128 // LANES):
                sh_v[r, pl.ds(j * LANES, LANES)] = z16
                dh_v[r, pl.ds(j * LANES, LANES)] = z16

        @pl.when(sid == 0)
        def _():
            pltpu.sync_copy(sh_v, ssh)
            pltpu.sync_copy(dh_v, dsh)

        pltpu.sync_copy(iota_h, iid_v)
        plsc.subcore_barrier()

        @pl.loop(0, per_tile)
        def _(ci):
            g = ci * NW + wid

            @pl.when(g < n_chunks)
            def _():
                pltpu.sync_copy(src_h.at[pl.ds(g * CHUNK, CHUNK)], sbuf)
                pltpu.sync_copy(dst_h.at[pl.ds(g * CHUNK, CHUNK)], dbuf)
                for j in range(CHUNK // LANES):
                    sv = sbuf[pl.ds(j * LANES, LANES)]
                    plsc.addupdate_scatter(
                        sh_v,
                        [lax.shift_right_logical(sv, 7),
                         lax.bitwise_and(sv, 127)],
                        ones16,
                    )
                    dv = dbuf[pl.ds(j * LANES, LANES)]
                    plsc.addupdate_scatter(
                        dh_v,
                        [lax.shift_right_logical(dv, 7),
                         lax.bitwise_and(dv, 127)],
                        ones16,
                    )

        pltpu.sync_copy(sh_v, ssh.at[iid_v.at[0]], add=True)
        pltpu.sync_copy(dh_v, dsh.at[iid_v.at[0]], add=True)
        plsc.subcore_barrier()

        @pl.when(sid == 0)
        def _():
            pltpu.sync_copy(ssh, out_h.at[cid, 0])
            pltpu.sync_copy(dsh, out_h.at[cid, 1])

    return k(src_flat, dst_flat, iota_rows)


def _sc_scatter(h2, dst2d, src_flat):
    n, d = h2.shape
    n_chunks = dst2d.shape[0]
    per_tile = (n_chunks + NS - 1) // NS
    rows_per = 640
    rows_last = n - rows_per * (NS - 1)
    mesh = plsc.VectorSubcoreMesh(core_axis_name="c", subcore_axis_name="s",
                                  num_cores=1)

    @functools.partial(
        pl.kernel,
        out_type=jax.ShapeDtypeStruct((n, d), jnp.float32),
        mesh=mesh,
        scratch_types=[
            pltpu.VMEM((CHUNK,), jnp.int32),
            pltpu.VMEM((1, CHUNK), jnp.int32),
            pltpu.VMEM((CHUNK, d), jnp.float32),
            pltpu.VMEM((16, d), jnp.float32),
            pltpu.VMEM_SHARED((n, d), jnp.float32),
            pltpu.SemaphoreType.DMA,
        ],
        compiler_params=_sc_compiler_params(),
    )
    def k(h2_h, d_h, s_h, out_h, sidx, didx, rows, zbuf, pooled, sem):
        sid = lax.axis_index("s")
        z16 = jnp.zeros((LANES,), jnp.float32)

        for i in range(16):
            for j in range(d // LANES):
                zbuf[i, pl.ds(j * LANES, LANES)] = z16

        r0 = sid * rows_per

        @pl.when(sid < NS - 1)
        def _():
            @pl.loop(0, rows_per // 16)
            def _(t):
                pltpu.sync_copy(zbuf, pooled.at[pl.ds(r0 + t * 16, 16)])

        @pl.when(sid == NS - 1)
        def _():
            @pl.loop(0, rows_last // 16)
            def _(t):
                pltpu.sync_copy(zbuf, pooled.at[pl.ds(r0 + t * 16, 16)])

        plsc.subcore_barrier()

        @pl.loop(0, per_tile)
        def _(ci):
            g = ci * NS + sid

            @pl.when(g < n_chunks)
            def _():
                pltpu.sync_copy(s_h.at[pl.ds(g * CHUNK, CHUNK)], sidx)
                pltpu.sync_copy(d_h.at[pl.ds(g, 1)], didx)
                pltpu.async_copy(h2_h.at[sidx], rows, sem).wait()
                pltpu.sync_copy(rows, pooled.at[didx.at[0]], add=True)

        plsc.subcore_barrier()

        @pl.when(sid < NS - 1)
        def _():
            pltpu.sync_copy(pooled.at[pl.ds(r0, rows_per)],
                            out_h.at[pl.ds(r0, rows_per)])

        @pl.when(sid == NS - 1)
        def _():
            pltpu.sync_copy(pooled.at[pl.ds(r0, rows_last)],
                            out_h.at[pl.ds(r0, rows_last)])

    return k(h2, dst2d, src_flat)


def _dot(a, b):
    return lax.dot_general(a, b, (((1,), (0,)), ((), ())),
                           precision=lax.Precision.HIGHEST,
                           preferred_element_type=jnp.float32)


def _mm_relu_body(x_ref, w_ref, b_ref, o_ref):
    o_ref[...] = jnp.maximum(_dot(x_ref[...], w_ref[...]) + b_ref[...], 0.0)


def _tc_mm_relu(x, w, b_row, block_rows):
    n, d = x.shape
    h = w.shape[1]
    grid = (n // block_rows,)
    return pl.pallas_call(
        _mm_relu_body,
        grid=grid,
        in_specs=[
            pl.BlockSpec((block_rows, d), lambda i: (i, 0)),
            pl.BlockSpec((d, h), lambda i: (0, 0)),
            pl.BlockSpec((1, h), lambda i: (0, 0)),
        ],
        out_specs=pl.BlockSpec((block_rows, h), lambda i: (i, 0)),
        out_shape=jax.ShapeDtypeStruct((n, h), jnp.float32),
    )(x, w, b_row)


def _scale_body(hist_ref, h_ref, h2_ref, rs_ref):
    out_deg = hist_ref[0, 0, :, :] + hist_ref[1, 0, :, :]
    in_deg = hist_ref[0, 1, :, :] + hist_ref[1, 1, :, :]
    rs_out = lax.rsqrt(jnp.maximum(out_deg, 1.0))
    h2_ref[...] = h_ref[...] * rs_out
    rs_ref[...] = lax.rsqrt(jnp.maximum(in_deg, 1.0))


def _tc_scale(hist_n, h_relu, block_rows):
    n, d = h_relu.shape
    grid = (n // block_rows,)
    return pl.pallas_call(
        _scale_body,
        grid=grid,
        in_specs=[
            pl.BlockSpec((NC, 2, block_rows, 1), lambda i: (0, 0, i, 0)),
            pl.BlockSpec((block_rows, d), lambda i: (i, 0)),
        ],
        out_specs=[
            pl.BlockSpec((block_rows, d), lambda i: (i, 0)),
            pl.BlockSpec((block_rows, 1), lambda i: (i, 0)),
        ],
        out_shape=[
            jax.ShapeDtypeStruct((n, d), jnp.float32),
            jax.ShapeDtypeStruct((n, 1), jnp.float32),
        ],
    )(hist_n, h_relu)


def _final_body(p_ref, rs_ref, w_ref, b_ref, x_ref, o_ref):
    s = p_ref[...] * rs_ref[...]
    o_ref[...] = jnp.maximum(_dot(s, w_ref[...]) + b_ref[...], 0.0) + x_ref[...]


def _tc_final(pooled, rs_col, w, b_row, x, block_rows):
    n, d = x.shape
    h = w.shape[1]
    grid = (n // block_rows,)
    return pl.pallas_call(
        _final_body,
        grid=grid,
        in_specs=[
            pl.BlockSpec((block_rows, h), lambda i: (i, 0)),
            pl.BlockSpec((block_rows, 1), lambda i: (i, 0)),
            pl.BlockSpec((h, h), lambda i: (0, 0)),
            pl.BlockSpec((1, h), lambda i: (0, 0)),
            pl.BlockSpec((block_rows, d), lambda i: (i, 0)),
        ],
        out_specs=pl.BlockSpec((block_rows, h), lambda i: (i, 0)),
        out_shape=jax.ShapeDtypeStruct((n, h), jnp.float32),
    )(pooled, rs_col, w, b_row, x)


def kernel(x, edge_index, W_edge, b_edge, W_node, b_node):
    x = x.astype(jnp.float32)
    ei = edge_index.astype(jnp.int32)
    src = ei[0]
    dst = ei[1]
    e = src.shape[0]
    n = x.shape[0]
    dst2d = dst.reshape(e // CHUNK, CHUNK)
    iota_rows = jnp.arange(HROWS, dtype=jnp.int32).reshape(1, HROWS)

    block_rows = 2000

    hist = _sc_degrees(src, dst, iota_rows)
    h_relu = _tc_mm_relu(x, W_edge, b_edge.reshape(1, -1), block_rows)
    hist_n = hist.reshape(NC, 2, HROWS * 128)[:, :, :n, None]
    h2, rs_col = _tc_scale(hist_n, h_relu, block_rows)
    pooled = _sc_scatter(h2, dst2d, src)
    out = _tc_final(pooled, rs_col, W_node, b_node.reshape(1, -1), x,
                    block_rows)
    return out


# speedup vs baseline: 18.3366x; 1.9187x over previous
import dataclasses
import functools

import jax
import jax.numpy as jnp
from jax import lax
from jax.experimental import pallas as pl
from jax.experimental.pallas import tpu as pltpu
from jax.experimental.pallas import tpu_sc as plsc

NC = 2
NS = 16
NW = NC * NS
LANES = 16
CHUNK = 128
HROWS = 80


def _sc_compiler_params():
    cp = pltpu.CompilerParams()
    if "needs_layout_passes" in pltpu.CompilerParams.__dataclass_fields__:
        cp = dataclasses.replace(cp, needs_layout_passes=False)
    return cp


def _sc_degrees(src_flat, dst_flat, iota_rows):
    e = src_flat.shape[0]
    e_per_tile = e // NW
    assert e_per_tile * NW == e and e_per_tile % LANES == 0
    mesh = plsc.VectorSubcoreMesh(core_axis_name="c", subcore_axis_name="s")

    @functools.partial(
        pl.kernel,
        out_type=jax.ShapeDtypeStruct((NC, 2, HROWS, 128), jnp.float32),
        mesh=mesh,
        scratch_types=[
            pltpu.VMEM((e_per_tile,), jnp.int32),
            pltpu.VMEM((e_per_tile,), jnp.int32),
            pltpu.VMEM((HROWS, 128), jnp.float32),
            pltpu.VMEM((HROWS, 128), jnp.float32),
            pltpu.VMEM((1, HROWS), jnp.int32),
            pltpu.VMEM_SHARED((HROWS, 128), jnp.float32),
            pltpu.VMEM_SHARED((HROWS, 128), jnp.float32),
        ],
        compiler_params=_sc_compiler_params(),
    )
    def k(src_h, dst_h, iota_h, out_h, sbuf, dbuf, sh_v, dh_v, iid_v, ssh, dsh):
        cid = lax.axis_index("c")
        sid = lax.axis_index("s")
        wid = sid * NC + cid
        z16 = jnp.zeros((LANES,), jnp.float32)
        ones16 = jnp.ones((LANES,), jnp.float32)

        @pl.loop(0, HROWS)
        def _(r):
            for j in range(128 // LANES):
                sh_v[r, pl.ds(j * LANES, LANES)] = z16
                dh_v[r, pl.ds(j * LANES, LANES)] = z16

        @pl.when(sid == 0)
        def _():
            pltpu.sync_copy(sh_v, ssh)
            pltpu.sync_copy(dh_v, dsh)

        pltpu.sync_copy(iota_h, iid_v)
        plsc.subcore_barrier()

        pltpu.sync_copy(src_h.at[pl.ds(wid * e_per_tile, e_per_tile)], sbuf)
        pltpu.sync_copy(dst_h.at[pl.ds(wid * e_per_tile, e_per_tile)], dbuf)

        @pl.loop(0, e_per_tile // LANES)
        def _(j):
            sv = sbuf[pl.ds(j * LANES, LANES)]
            plsc.addupdate_scatter(
                sh_v,
                [lax.shift_right_logical(sv, 7), lax.bitwise_and(sv, 127)],
                ones16,
            )
            dv = dbuf[pl.ds(j * LANES, LANES)]
            plsc.addupdate_scatter(
                dh_v,
                [lax.shift_right_logical(dv, 7), lax.bitwise_and(dv, 127)],
                ones16,
            )

        pltpu.sync_copy(sh_v, ssh.at[iid_v.at[0]], add=True)
        pltpu.sync_copy(dh_v, dsh.at[iid_v.at[0]], add=True)
        plsc.subcore_barrier()

        @pl.when(sid == 0)
        def _():
            pltpu.sync_copy(ssh, out_h.at[cid, 0])
            pltpu.sync_copy(dsh, out_h.at[cid, 1])

    return k(src_flat, dst_flat, iota_rows)


DUMP = 64
BLK = 8


def _sc_scatter(h2, dst2d, src_flat):
    n, d = h2.shape
    n_chunks = dst2d.shape[0]
    per_tile = n_chunks // NS
    blocks = per_tile // BLK
    rows_per = 640
    rows_last = n - rows_per * (NS - 1)
    mesh = plsc.VectorSubcoreMesh(core_axis_name="c", subcore_axis_name="s",
                                  num_cores=1)

    @functools.partial(
        pl.kernel,
        out_type=jax.ShapeDtypeStruct((n, d), jnp.float32),
        mesh=mesh,
        scratch_types=[
            pltpu.VMEM((BLK * CHUNK,), jnp.int32),
            pltpu.VMEM((BLK, CHUNK), jnp.int32),
            pltpu.VMEM((2, CHUNK, d), jnp.float32),
            pltpu.VMEM((16, d), jnp.float32),
            pltpu.VMEM_SHARED((n + DUMP, d), jnp.float32),
            pltpu.SemaphoreType.DMA,
            pltpu.SemaphoreType.DMA,
            pltpu.SemaphoreType.DMA,
            pltpu.SemaphoreType.DMA,
        ],
        compiler_params=_sc_compiler_params(),
    )
    def k(h2_h, d_h, s_h, out_h, sblk, dblk, rows, zbuf, pooled, g0, g1, s0, s1):
        sid = lax.axis_index("s")
        gsem = (g0, g1)
        ssem = (s0, s1)
        z16 = jnp.zeros((LANES,), jnp.float32)

        for i in range(16):
            for j in range(d // LANES):
                zbuf[i, pl.ds(j * LANES, LANES)] = z16

        r0 = sid * rows_per

        @pl.when(sid < NS - 1)
        def _():
            @pl.loop(0, rows_per // 16)
            def _(t):
                pltpu.sync_copy(zbuf, pooled.at[pl.ds(r0 + t * 16, 16)])

        @pl.when(sid == NS - 1)
        def _():
            @pl.loop(0, rows_last // 16)
            def _(t):
                pltpu.sync_copy(zbuf, pooled.at[pl.ds(r0 + t * 16, 16)])

        plsc.subcore_barrier()

        base_chunk = sid * per_tile

        @pl.loop(0, blocks)
        def _(b):
            c0 = base_chunk + b * BLK
            pltpu.sync_copy(s_h.at[pl.ds(c0 * CHUNK, BLK * CHUNK)], sblk)
            pltpu.sync_copy(d_h.at[pl.ds(c0, BLK)], dblk)
            pend_g = [None, None]
            pend_s = [None, None]
            for j in range(BLK):
                slot = j & 1
                if pend_s[slot] is not None:
                    pend_s[slot].wait()
                pend_g[slot] = pltpu.async_copy(
                    h2_h.at[sblk.at[pl.ds(j * CHUNK, CHUNK)]],
                    rows.at[slot], gsem[slot])
                if j >= 1:
                    other = slot ^ 1
                    pend_g[other].wait()
                    pend_s[other] = pltpu.async_copy(
                        rows.at[other], pooled.at[dblk.at[j - 1]],
                        ssem[other], add=True)
            last = (BLK - 1) & 1
            pend_g[last].wait()
            pend_s[last] = pltpu.async_copy(
                rows.at[last], pooled.at[dblk.at[BLK - 1]], ssem[last],
                add=True)
            pend_s[last ^ 1].wait()
            pend_s[last].wait()

        plsc.subcore_barrier()

        @pl.when(sid < NS - 1)
        def _():
            pltpu.sync_copy(pooled.at[pl.ds(r0, rows_per)],
                            out_h.at[pl.ds(r0, rows_per)])

        @pl.when(sid == NS - 1)
        def _():
            pltpu.sync_copy(pooled.at[pl.ds(r0, rows_last)],
                            out_h.at[pl.ds(r0, rows_last)])

    return k(h2, dst2d, src_flat)


def _dot(a, b):
    return lax.dot_general(a, b, (((1,), (0,)), ((), ())),
                           precision=lax.Precision.HIGHEST,
                           preferred_element_type=jnp.float32)


def _mm_relu_body(x_ref, w_ref, b_ref, o_ref):
    o_ref[...] = jnp.maximum(_dot(x_ref[...], w_ref[...]) + b_ref[...], 0.0)


def _tc_mm_relu(x, w, b_row, block_rows):
    n, d = x.shape
    h = w.shape[1]
    grid = (n // block_rows,)
    return pl.pallas_call(
        _mm_relu_body,
        grid=grid,
        in_specs=[
            pl.BlockSpec((block_rows, d), lambda i: (i, 0)),
            pl.BlockSpec((d, h), lambda i: (0, 0)),
            pl.BlockSpec((1, h), lambda i: (0, 0)),
        ],
        out_specs=pl.BlockSpec((block_rows, h), lambda i: (i, 0)),
        out_shape=jax.ShapeDtypeStruct((n, h), jnp.float32),
    )(x, w, b_row)


def _scale_body(hist_ref, h_ref, h2_ref, rs_ref):
    out_deg = hist_ref[0, 0, :, :] + hist_ref[1, 0, :, :]
    in_deg = hist_ref[0, 1, :, :] + hist_ref[1, 1, :, :]
    rs_out = lax.rsqrt(jnp.maximum(out_deg, 1.0))
    h2_ref[...] = h_ref[...] * rs_out
    rs_ref[...] = lax.rsqrt(jnp.maximum(in_deg, 1.0))


def _tc_scale(hist_n, h_relu, block_rows):
    n, d = h_relu.shape
    grid = (n // block_rows,)
    return pl.pallas_call(
        _scale_body,
        grid=grid,
        in_specs=[
            pl.BlockSpec((NC, 2, block_rows, 1), lambda i: (0, 0, i, 0)),
            pl.BlockSpec((block_rows, d), lambda i: (i, 0)),
        ],
        out_specs=[
            pl.BlockSpec((block_rows, d), lambda i: (i, 0)),
            pl.BlockSpec((block_rows, 1), lambda i: (i, 0)),
        ],
        out_shape=[
            jax.ShapeDtypeStruct((n, d), jnp.float32),
            jax.ShapeDtypeStruct((n, 1), jnp.float32),
        ],
    )(hist_n, h_relu)


def _final_body(p_ref, rs_ref, w_ref, b_ref, x_ref, o_ref):
    s = p_ref[...] * rs_ref[...]
    o_ref[...] = jnp.maximum(_dot(s, w_ref[...]) + b_ref[...], 0.0) + x_ref[...]


def _tc_final(pooled, rs_col, w, b_row, x, block_rows):
    n, d = x.shape
    h = w.shape[1]
    grid = (n // block_rows,)
    return pl.pallas_call(
        _final_body,
        grid=grid,
        in_specs=[
            pl.BlockSpec((block_rows, h), lambda i: (i, 0)),
            pl.BlockSpec((block_rows, 1), lambda i: (i, 0)),
            pl.BlockSpec((h, h), lambda i: (0, 0)),
            pl.BlockSpec((1, h), lambda i: (0, 0)),
            pl.BlockSpec((block_rows, d), lambda i: (i, 0)),
        ],
        out_specs=pl.BlockSpec((block_rows, h), lambda i: (i, 0)),
        out_shape=jax.ShapeDtypeStruct((n, h), jnp.float32),
    )(pooled, rs_col, w, b_row, x)


def kernel(x, edge_index, W_edge, b_edge, W_node, b_node):
    x = x.astype(jnp.float32)
    ei = edge_index.astype(jnp.int32)
    src = ei[0]
    dst = ei[1]
    e = src.shape[0]
    n = x.shape[0]
    chunks_pad = -(-e // (CHUNK * NS * BLK)) * NS * BLK
    pad_e = chunks_pad * CHUNK - e
    pad_ar = jnp.arange(pad_e, dtype=jnp.int32)
    src_p = jnp.concatenate([src, pad_ar & 1023])
    dst_p = jnp.concatenate([dst, n + (pad_ar & (DUMP - 1))])
    dst2d = dst_p.reshape(chunks_pad, CHUNK)
    iota_rows = jnp.arange(HROWS, dtype=jnp.int32).reshape(1, HROWS)

    block_rows = 2000

    hist = _sc_degrees(src, dst, iota_rows)
    h_relu = _tc_mm_relu(x, W_edge, b_edge.reshape(1, -1), block_rows)
    hist_n = hist.reshape(NC, 2, HROWS * 128)[:, :, :n, None]
    h2, rs_col = _tc_scale(hist_n, h_relu, block_rows)
    pooled = _sc_scatter(h2, dst2d, src_p)
    out = _tc_final(pooled, rs_col, W_node, b_node.reshape(1, -1), x,
                    block_rows)
    return out


# speedup vs baseline: 20.4090x; 1.1130x over previous
import dataclasses
import functools

import jax
import jax.numpy as jnp
from jax import lax
from jax.experimental import pallas as pl
from jax.experimental.pallas import tpu as pltpu
from jax.experimental.pallas import tpu_sc as plsc

NC = 2
NS = 16
NW = NC * NS
LANES = 16
CHUNK = 128
HROWS = 80


def _sc_compiler_params():
    cp = pltpu.CompilerParams()
    if "needs_layout_passes" in pltpu.CompilerParams.__dataclass_fields__:
        cp = dataclasses.replace(cp, needs_layout_passes=False)
    return cp


def _sc_degrees(src_flat, dst_flat, iota_rows):
    e = src_flat.shape[0]
    e_per_tile = e // NW
    assert e_per_tile * NW == e and e_per_tile % LANES == 0
    mesh = plsc.VectorSubcoreMesh(core_axis_name="c", subcore_axis_name="s")

    @functools.partial(
        pl.kernel,
        out_type=jax.ShapeDtypeStruct((NC, 2, HROWS, 128), jnp.float32),
        mesh=mesh,
        scratch_types=[
            pltpu.VMEM((e_per_tile,), jnp.int32),
            pltpu.VMEM((e_per_tile,), jnp.int32),
            pltpu.VMEM((HROWS, 128), jnp.float32),
            pltpu.VMEM((HROWS, 128), jnp.float32),
            pltpu.VMEM((1, HROWS), jnp.int32),
            pltpu.VMEM_SHARED((HROWS, 128), jnp.float32),
            pltpu.VMEM_SHARED((HROWS, 128), jnp.float32),
        ],
        compiler_params=_sc_compiler_params(),
    )
    def k(src_h, dst_h, iota_h, out_h, sbuf, dbuf, sh_v, dh_v, iid_v, ssh, dsh):
        cid = lax.axis_index("c")
        sid = lax.axis_index("s")
        wid = sid * NC + cid
        z16 = jnp.zeros((LANES,), jnp.float32)
        ones16 = jnp.ones((LANES,), jnp.float32)

        @pl.loop(0, HROWS)
        def _(r):
            for j in range(128 // LANES):
                sh_v[r, pl.ds(j * LANES, LANES)] = z16
                dh_v[r, pl.ds(j * LANES, LANES)] = z16

        @pl.when(sid == 0)
        def _():
            pltpu.sync_copy(sh_v, ssh)
            pltpu.sync_copy(dh_v, dsh)

        pltpu.sync_copy(iota_h, iid_v)
        plsc.subcore_barrier()

        pltpu.sync_copy(src_h.at[pl.ds(wid * e_per_tile, e_per_tile)], sbuf)
        pltpu.sync_copy(dst_h.at[pl.ds(wid * e_per_tile, e_per_tile)], dbuf)

        @pl.loop(0, e_per_tile // LANES)
        def _(j):
            sv = sbuf[pl.ds(j * LANES, LANES)]
            plsc.addupdate_scatter(
                sh_v,
                [lax.shift_right_logical(sv, 7), lax.bitwise_and(sv, 127)],
                ones16,
            )
            dv = dbuf[pl.ds(j * LANES, LANES)]
            plsc.addupdate_scatter(
                dh_v,
                [lax.shift_right_logical(dv, 7), lax.bitwise_and(dv, 127)],
                ones16,
            )

        pltpu.sync_copy(sh_v, ssh.at[iid_v.at[0]], add=True)
        pltpu.sync_copy(dh_v, dsh.at[iid_v.at[0]], add=True)
        plsc.subcore_barrier()

        @pl.when(sid == 0)
        def _():
            pltpu.sync_copy(ssh, out_h.at[cid, 0])
            pltpu.sync_copy(dsh, out_h.at[cid, 1])

    return k(src_flat, dst_flat, iota_rows)


DUMP = 64
BLK = 16


def _sc_scatter(h2, dst2d, src_flat):
    n, d = h2.shape
    n_chunks = dst2d.shape[0]
    per_tile = n_chunks // NS
    blocks = per_tile // BLK
    rows_per = 640
    rows_last = n - rows_per * (NS - 1)
    mesh = plsc.VectorSubcoreMesh(core_axis_name="c", subcore_axis_name="s",
                                  num_cores=1)

    @functools.partial(
        pl.kernel,
        out_type=jax.ShapeDtypeStruct((n, d), jnp.float32),
        mesh=mesh,
        scratch_types=[
            pltpu.VMEM((2, BLK * CHUNK), jnp.int32),
            pltpu.VMEM((2, BLK, CHUNK), jnp.int32),
            pltpu.VMEM((2, CHUNK, d), jnp.float32),
            pltpu.VMEM((16, d), jnp.float32),
            pltpu.VMEM_SHARED((n + DUMP, d), jnp.float32),
            pltpu.SemaphoreType.DMA,
            pltpu.SemaphoreType.DMA,
            pltpu.SemaphoreType.DMA,
            pltpu.SemaphoreType.DMA,
            pltpu.SemaphoreType.DMA,
        ],
        compiler_params=_sc_compiler_params(),
    )
    def k(h2_h, d_h, s_h, out_h, sblk, dblk, rows, zbuf, pooled,
          g0, g1, s0, s1, isem):
        sid = lax.axis_index("s")
        gsem = (g0, g1)
        ssem = (s0, s1)
        z16 = jnp.zeros((LANES,), jnp.float32)

        for i in range(16):
            for j in range(d // LANES):
                zbuf[i, pl.ds(j * LANES, LANES)] = z16

        r0 = sid * rows_per

        @pl.when(sid < NS - 1)
        def _():
            @pl.loop(0, rows_per // 16)
            def _(t):
                pltpu.sync_copy(zbuf, pooled.at[pl.ds(r0 + t * 16, 16)])

        @pl.when(sid == NS - 1)
        def _():
            @pl.loop(0, rows_last // 16)
            def _(t):
                pltpu.sync_copy(zbuf, pooled.at[pl.ds(r0 + t * 16, 16)])

        plsc.subcore_barrier()

        base_chunk = sid * per_tile

        pltpu.sync_copy(s_h.at[pl.ds(base_chunk * CHUNK, BLK * CHUNK)],
                        sblk.at[0])
        pltpu.sync_copy(d_h.at[pl.ds(base_chunk, BLK)], dblk.at[0])

        @pl.loop(0, blocks)
        def _(b):
            ib = lax.rem(b, 2)
            ibn = 1 - ib
            nxt = lax.min(b + 1, blocks - 1)
            c1 = base_chunk + nxt * BLK
            pi1 = pltpu.async_copy(
                s_h.at[pl.ds(c1 * CHUNK, BLK * CHUNK)], sblk.at[ibn], isem)
            pi2 = pltpu.async_copy(d_h.at[pl.ds(c1, BLK)], dblk.at[ibn], isem)
            pend_g = [None, None]
            pend_s = [None, None]
            for j in range(BLK):
                slot = j & 1
                if pend_s[slot] is not None:
                    pend_s[slot].wait()
                pend_g[slot] = pltpu.async_copy(
                    h2_h.at[sblk.at[ib, pl.ds(j * CHUNK, CHUNK)]],
                    rows.at[slot], gsem[slot])
                if j >= 1:
                    other = slot ^ 1
                    pend_g[other].wait()
                    pend_s[other] = pltpu.async_copy(
                        rows.at[other], pooled.at[dblk.at[ib, j - 1]],
                        ssem[other], add=True)
            last = (BLK - 1) & 1
            pend_g[last].wait()
            pend_s[last] = pltpu.async_copy(
                rows.at[last], pooled.at[dblk.at[ib, BLK - 1]], ssem[last],
                add=True)
            pend_s[last ^ 1].wait()
            pend_s[last].wait()
            pi1.wait()
            pi2.wait()

        plsc.subcore_barrier()

        @pl.when(sid < NS - 1)
        def _():
            pltpu.sync_copy(pooled.at[pl.ds(r0, rows_per)],
                            out_h.at[pl.ds(r0, rows_per)])

        @pl.when(sid == NS - 1)
        def _():
            pltpu.sync_copy(pooled.at[pl.ds(r0, rows_last)],
                            out_h.at[pl.ds(r0, rows_last)])

    return k(h2, dst2d, src_flat)


def _dot(a, b):
    return lax.dot_general(a, b, (((1,), (0,)), ((), ())),
                           precision=lax.Precision.HIGHEST,
                           preferred_element_type=jnp.float32)


def _mm_scale_body(hist_ref, x_ref, w_ref, b_ref, h2_ref, rs_ref):
    out_deg = hist_ref[0, 0, :, :] + hist_ref[1, 0, :, :]
    in_deg = hist_ref[0, 1, :, :] + hist_ref[1, 1, :, :]
    rs_out = lax.rsqrt(jnp.maximum(out_deg, 1.0))
    h_relu = jnp.maximum(_dot(x_ref[...], w_ref[...]) + b_ref[...], 0.0)
    h2_ref[...] = h_relu * rs_out
    rs_ref[...] = lax.rsqrt(jnp.maximum(in_deg, 1.0))


def _tc_mm_scale(hist_n, x, w, b_row, block_rows):
    n, d = x.shape
    h = w.shape[1]
    grid = (n // block_rows,)
    return pl.pallas_call(
        _mm_scale_body,
        grid=grid,
        in_specs=[
            pl.BlockSpec((NC, 2, block_rows, 1), lambda i: (0, 0, i, 0)),
            pl.BlockSpec((block_rows, d), lambda i: (i, 0)),
            pl.BlockSpec((d, h), lambda i: (0, 0)),
            pl.BlockSpec((1, h), lambda i: (0, 0)),
        ],
        out_specs=[
            pl.BlockSpec((block_rows, h), lambda i: (i, 0)),
            pl.BlockSpec((block_rows, 1), lambda i: (i, 0)),
        ],
        out_shape=[
            jax.ShapeDtypeStruct((n, h), jnp.float32),
            jax.ShapeDtypeStruct((n, 1), jnp.float32),
        ],
    )(hist_n, x, w, b_row)


def _final_body(p_ref, rs_ref, w_ref, b_ref, x_ref, o_ref):
    s = p_ref[...] * rs_ref[...]
    o_ref[...] = jnp.maximum(_dot(s, w_ref[...]) + b_ref[...], 0.0) + x_ref[...]


def _tc_final(pooled, rs_col, w, b_row, x, block_rows):
    n, d = x.shape
    h = w.shape[1]
    grid = (n // block_rows,)
    return pl.pallas_call(
        _final_body,
        grid=grid,
        in_specs=[
            pl.BlockSpec((block_rows, h), lambda i: (i, 0)),
            pl.BlockSpec((block_rows, 1), lambda i: (i, 0)),
            pl.BlockSpec((h, h), lambda i: (0, 0)),
            pl.BlockSpec((1, h), lambda i: (0, 0)),
            pl.BlockSpec((block_rows, d), lambda i: (i, 0)),
        ],
        out_specs=pl.BlockSpec((block_rows, h), lambda i: (i, 0)),
        out_shape=jax.ShapeDtypeStruct((n, h), jnp.float32),
    )(pooled, rs_col, w, b_row, x)


def kernel(x, edge_index, W_edge, b_edge, W_node, b_node):
    x = x.astype(jnp.float32)
    ei = edge_index.astype(jnp.int32)
    src = ei[0]
    dst = ei[1]
    e = src.shape[0]
    n = x.shape[0]
    chunks_pad = -(-e // (CHUNK * NS * BLK)) * NS * BLK
    pad_e = chunks_pad * CHUNK - e
    pad_ar = jnp.arange(pad_e, dtype=jnp.int32)
    src_p = jnp.concatenate([src, pad_ar & 1023])
    dst_p = jnp.concatenate([dst, n + (pad_ar & (DUMP - 1))])
    dst2d = dst_p.reshape(chunks_pad, CHUNK)
    iota_rows = jnp.arange(HROWS, dtype=jnp.int32).reshape(1, HROWS)

    block_rows = 2000

    hist = _sc_degrees(src, dst, iota_rows)
    hist_n = hist.reshape(NC, 2, HROWS * 128)[:, :, :n, None]
    h2, rs_col = _tc_mm_scale(hist_n, x, W_edge, b_edge.reshape(1, -1),
                              block_rows)
    pooled = _sc_scatter(h2, dst2d, src_p)
    out = _tc_final(pooled, rs_col, W_node, b_node.reshape(1, -1), x,
                    block_rows)
    return out


# speedup vs baseline: 21.1382x; 1.0357x over previous
import dataclasses
import functools

import jax
import jax.numpy as jnp
from jax import lax
from jax.experimental import pallas as pl
from jax.experimental.pallas import tpu as pltpu
from jax.experimental.pallas import tpu_sc as plsc

NC = 2
NS = 16
NW = NC * NS
LANES = 16
CHUNK = 128
HROWS = 80


def _sc_compiler_params():
    cp = pltpu.CompilerParams()
    if "needs_layout_passes" in pltpu.CompilerParams.__dataclass_fields__:
        cp = dataclasses.replace(cp, needs_layout_passes=False)
    return cp


def _sc_degrees(src_flat, dst_flat, iota_rows):
    e = src_flat.shape[0]
    e_per_tile = e // NW
    assert e_per_tile * NW == e and e_per_tile % LANES == 0
    mesh = plsc.VectorSubcoreMesh(core_axis_name="c", subcore_axis_name="s")

    @functools.partial(
        pl.kernel,
        out_type=jax.ShapeDtypeStruct((NC, 2, HROWS, 128), jnp.float32),
        mesh=mesh,
        scratch_types=[
            pltpu.VMEM((e_per_tile,), jnp.int32),
            pltpu.VMEM((e_per_tile,), jnp.int32),
            pltpu.VMEM((HROWS, 128), jnp.float32),
            pltpu.VMEM((HROWS, 128), jnp.float32),
            pltpu.VMEM((1, HROWS), jnp.int32),
            pltpu.VMEM_SHARED((HROWS, 128), jnp.float32),
            pltpu.VMEM_SHARED((HROWS, 128), jnp.float32),
        ],
        compiler_params=_sc_compiler_params(),
    )
    def k(src_h, dst_h, iota_h, out_h, sbuf, dbuf, sh_v, dh_v, iid_v, ssh, dsh):
        cid = lax.axis_index("c")
        sid = lax.axis_index("s")
        wid = sid * NC + cid
        z16 = jnp.zeros((LANES,), jnp.float32)
        ones16 = jnp.ones((LANES,), jnp.float32)

        @pl.loop(0, HROWS)
        def _(r):
            for j in range(128 // LANES):
                sh_v[r, pl.ds(j * LANES, LANES)] = z16
                dh_v[r, pl.ds(j * LANES, LANES)] = z16

        @pl.when(sid == 0)
        def _():
            pltpu.sync_copy(sh_v, ssh)
            pltpu.sync_copy(dh_v, dsh)

        pltpu.sync_copy(iota_h, iid_v)
        plsc.subcore_barrier()

        pltpu.sync_copy(src_h.at[pl.ds(wid * e_per_tile, e_per_tile)], sbuf)
        pltpu.sync_copy(dst_h.at[pl.ds(wid * e_per_tile, e_per_tile)], dbuf)

        @pl.loop(0, e_per_tile // LANES)
        def _(j):
            sv = sbuf[pl.ds(j * LANES, LANES)]
            plsc.addupdate_scatter(
                sh_v,
                [lax.shift_right_logical(sv, 7), lax.bitwise_and(sv, 127)],
                ones16,
            )
            dv = dbuf[pl.ds(j * LANES, LANES)]
            plsc.addupdate_scatter(
                dh_v,
                [lax.shift_right_logical(dv, 7), lax.bitwise_and(dv, 127)],
                ones16,
            )

        pltpu.sync_copy(sh_v, ssh.at[iid_v.at[0]], add=True)
        pltpu.sync_copy(dh_v, dsh.at[iid_v.at[0]], add=True)
        plsc.subcore_barrier()

        @pl.when(sid == 0)
        def _():
            pltpu.sync_copy(ssh, out_h.at[cid, 0])
            pltpu.sync_copy(dsh, out_h.at[cid, 1])

    return k(src_flat, dst_flat, iota_rows)


DUMP = 64
BLK = 16


def _sc_scatter(h2, dst2d, src_flat):
    n, d = h2.shape
    n_chunks = dst2d.shape[0]
    per_tile = n_chunks // NS
    blocks = per_tile // BLK
    rows_per = 640
    rows_last = n - rows_per * (NS - 1)
    mesh = plsc.VectorSubcoreMesh(core_axis_name="c", subcore_axis_name="s",
                                  num_cores=1)

    @functools.partial(
        pl.kernel,
        out_type=jax.ShapeDtypeStruct((n, d), jnp.float32),
        mesh=mesh,
        scratch_types=[
            pltpu.VMEM((2, BLK * CHUNK), jnp.int32),
            pltpu.VMEM((2, BLK, CHUNK), jnp.int32),
            pltpu.VMEM((2, CHUNK, d), jnp.float32),
            pltpu.VMEM((16, d), jnp.float32),
            pltpu.VMEM_SHARED((n + DUMP, d), jnp.float32),
            pltpu.SemaphoreType.DMA,
            pltpu.SemaphoreType.DMA,
            pltpu.SemaphoreType.DMA,
            pltpu.SemaphoreType.DMA,
            pltpu.SemaphoreType.DMA,
        ],
        compiler_params=_sc_compiler_params(),
    )
    def k(h2_h, d_h, s_h, out_h, sblk, dblk, rows, zbuf, pooled,
          g0, g1, s0, s1, isem):
        sid = lax.axis_index("s")
        gsem = (g0, g1)
        ssem = (s0, s1)
        z16 = jnp.zeros((LANES,), jnp.float32)

        for i in range(16):
            for j in range(d // LANES):
                zbuf[i, pl.ds(j * LANES, LANES)] = z16

        r0 = sid * rows_per

        @pl.when(sid < NS - 1)
        def _():
            @pl.loop(0, rows_per // 16)
            def _(t):
                pltpu.sync_copy(zbuf, pooled.at[pl.ds(r0 + t * 16, 16)])

        @pl.when(sid == NS - 1)
        def _():
            @pl.loop(0, rows_last // 16)
            def _(t):
                pltpu.sync_copy(zbuf, pooled.at[pl.ds(r0 + t * 16, 16)])

        plsc.subcore_barrier()

        base_chunk = sid * per_tile

        pltpu.sync_copy(s_h.at[pl.ds(base_chunk * CHUNK, BLK * CHUNK)],
                        sblk.at[0])
        pltpu.sync_copy(d_h.at[pl.ds(base_chunk, BLK)], dblk.at[0])

        pend_g = [None, None]
        pend_s = [None, None]
        pend_pi = None
        for b in range(blocks):
            ib = b & 1
            if pend_pi is not None:
                pend_pi[0].wait()
                pend_pi[1].wait()
                pend_pi = None
            for j in range(BLK):
                slot = j & 1
                if pend_s[slot] is not None:
                    pend_s[slot].wait()
                pend_g[slot] = pltpu.async_copy(
                    h2_h.at[sblk.at[ib, pl.ds(j * CHUNK, CHUNK)]],
                    rows.at[slot], gsem[slot])
                t = b * BLK + j
                if t >= 1:
                    other = slot ^ 1
                    jprev = j - 1 if j >= 1 else BLK - 1
                    ibprev = ib if j >= 1 else ib ^ 1
                    pend_g[other].wait()
                    pend_s[other] = pltpu.async_copy(
                        rows.at[other], pooled.at[dblk.at[ibprev, jprev]],
                        ssem[other], add=True)
                if j == 2 and b + 1 < blocks:
                    c1 = base_chunk + (b + 1) * BLK
                    pend_pi = (
                        pltpu.async_copy(
                            s_h.at[pl.ds(c1 * CHUNK, BLK * CHUNK)],
                            sblk.at[ib ^ 1], isem),
                        pltpu.async_copy(
                            d_h.at[pl.ds(c1, BLK)], dblk.at[ib ^ 1], isem),
                    )
        last = (BLK - 1) & 1
        lb = (blocks - 1) & 1
        pend_g[last].wait()
        pend_s[last] = pltpu.async_copy(
            rows.at[last], pooled.at[dblk.at[lb, BLK - 1]], ssem[last],
            add=True)
        pend_s[last ^ 1].wait()
        pend_s[last].wait()

        plsc.subcore_barrier()

        @pl.when(sid < NS - 1)
        def _():
            pltpu.sync_copy(pooled.at[pl.ds(r0, rows_per)],
                            out_h.at[pl.ds(r0, rows_per)])

        @pl.when(sid == NS - 1)
        def _():
            pltpu.sync_copy(pooled.at[pl.ds(r0, rows_last)],
                            out_h.at[pl.ds(r0, rows_last)])

    return k(h2, dst2d, src_flat)


def _dot(a, b):
    return lax.dot_general(a, b, (((1,), (0,)), ((), ())),
                           precision=lax.Precision.HIGHEST,
                           preferred_element_type=jnp.float32)


def _mm_scale_body(hist_ref, x_ref, w_ref, b_ref, h2_ref, rs_ref):
    out_deg = hist_ref[0, 0, :, :] + hist_ref[1, 0, :, :]
    in_deg = hist_ref[0, 1, :, :] + hist_ref[1, 1, :, :]
    rs_out = lax.rsqrt(jnp.maximum(out_deg, 1.0))
    h_relu = jnp.maximum(_dot(x_ref[...], w_ref[...]) + b_ref[...], 0.0)
    h2_ref[...] = h_relu * rs_out
    rs_ref[...] = lax.rsqrt(jnp.maximum(in_deg, 1.0))


def _tc_mm_scale(hist_n, x, w, b_row, block_rows):
    n, d = x.shape
    h = w.shape[1]
    grid = (n // block_rows,)
    return pl.pallas_call(
        _mm_scale_body,
        grid=grid,
        in_specs=[
            pl.BlockSpec((NC, 2, block_rows, 1), lambda i: (0, 0, i, 0)),
            pl.BlockSpec((block_rows, d), lambda i: (i, 0)),
            pl.BlockSpec((d, h), lambda i: (0, 0)),
            pl.BlockSpec((1, h), lambda i: (0, 0)),
        ],
        out_specs=[
            pl.BlockSpec((block_rows, h), lambda i: (i, 0)),
            pl.BlockSpec((block_rows, 1), lambda i: (i, 0)),
        ],
        out_shape=[
            jax.ShapeDtypeStruct((n, h), jnp.float32),
            jax.ShapeDtypeStruct((n, 1), jnp.float32),
        ],
    )(hist_n, x, w, b_row)


def _final_body(p_ref, rs_ref, w_ref, b_ref, x_ref, o_ref):
    s = p_ref[...] * rs_ref[...]
    o_ref[...] = jnp.maximum(_dot(s, w_ref[...]) + b_ref[...], 0.0) + x_ref[...]


def _tc_final(pooled, rs_col, w, b_row, x, block_rows):
    n, d = x.shape
    h = w.shape[1]
    grid = (n // block_rows,)
    return pl.pallas_call(
        _final_body,
        grid=grid,
        in_specs=[
            pl.BlockSpec((block_rows, h), lambda i: (i, 0)),
            pl.BlockSpec((block_rows, 1), lambda i: (i, 0)),
            pl.BlockSpec((h, h), lambda i: (0, 0)),
            pl.BlockSpec((1, h), lambda i: (0, 0)),
            pl.BlockSpec((block_rows, d), lambda i: (i, 0)),
        ],
        out_specs=pl.BlockSpec((block_rows, h), lambda i: (i, 0)),
        out_shape=jax.ShapeDtypeStruct((n, h), jnp.float32),
    )(pooled, rs_col, w, b_row, x)


def kernel(x, edge_index, W_edge, b_edge, W_node, b_node):
    x = x.astype(jnp.float32)
    ei = edge_index.astype(jnp.int32)
    src = ei[0]
    dst = ei[1]
    e = src.shape[0]
    n = x.shape[0]
    chunks_pad = -(-e // (CHUNK * NS * BLK)) * NS * BLK
    pad_e = chunks_pad * CHUNK - e
    pad_ar = jnp.arange(pad_e, dtype=jnp.int32)
    src_p = jnp.concatenate([src, pad_ar & 1023])
    dst_p = jnp.concatenate([dst, n + (pad_ar & (DUMP - 1))])
    dst2d = dst_p.reshape(chunks_pad, CHUNK)
    iota_rows = jnp.arange(HROWS, dtype=jnp.int32).reshape(1, HROWS)

    block_rows = 2000

    hist = _sc_degrees(src, dst, iota_rows)
    hist_n = hist.reshape(NC, 2, HROWS * 128)[:, :, :n, None]
    h2, rs_col = _tc_mm_scale(hist_n, x, W_edge, b_edge.reshape(1, -1),
                              block_rows)
    pooled = _sc_scatter(h2, dst2d, src_p)
    out = _tc_final(pooled, rs_col, W_node, b_node.reshape(1, -1), x,
                    block_rows)
    return out


# speedup vs baseline: 22.2761x; 1.0538x over previous
import dataclasses
import functools

import jax
import jax.numpy as jnp
from jax import lax
from jax.experimental import pallas as pl
from jax.experimental.pallas import tpu as pltpu
from jax.experimental.pallas import tpu_sc as plsc

NC = 2
NS = 16
NW = NC * NS
LANES = 16
HROWS = 80
CHUNK = 64
BLK = 8
DUMP = 64
CHK = 2000


def _sc_compiler_params():
    cp = pltpu.CompilerParams()
    if "needs_layout_passes" in pltpu.CompilerParams.__dataclass_fields__:
        cp = dataclasses.replace(cp, needs_layout_passes=False)
    return cp


def _sc_degrees_partition(src_flat, dst_flat, iota_rows, n):
    e = src_flat.shape[0]
    ept = e // NS
    assert ept * NS == e and ept % CHK == 0
    half = n // 2
    slot = (ept + 2 * BLK * CHUNK - 1) // (BLK * CHUNK) * (BLK * CHUNK)
    mesh = plsc.VectorSubcoreMesh(core_axis_name="c", subcore_axis_name="s")

    @functools.partial(
        pl.kernel,
        out_type=[
            jax.ShapeDtypeStruct((NC, 2, HROWS, 128), jnp.float32),
            jax.ShapeDtypeStruct((NC, NS, slot), jnp.int32),
            jax.ShapeDtypeStruct((NC, NS, slot), jnp.int32),
            jax.ShapeDtypeStruct((NC, NS, 16), jnp.int32),
        ],
        mesh=mesh,
        scratch_types=[
            pltpu.VMEM((CHK,), jnp.int32),
            pltpu.VMEM((CHK,), jnp.int32),
            pltpu.VMEM((HROWS, 128), jnp.float32),
            pltpu.VMEM((slot,), jnp.int32),
            pltpu.VMEM((slot,), jnp.int32),
            pltpu.VMEM((1, HROWS), jnp.int32),
            pltpu.VMEM((16,), jnp.int32),
            pltpu.VMEM_SHARED((HROWS, 128), jnp.float32),
            pltpu.VMEM_SHARED((HROWS, 128), jnp.float32),
        ],
        compiler_params=_sc_compiler_params(),
    )
    def k(src_h, dst_h, iota_h, hist_o, srcl_o, dstl_o, cnt_o,
          sbuf, dbuf, hist_v, srcv, dstv, iid_v, cntv, ssh, dsh):
        cid = lax.axis_index("c")
        sid = lax.axis_index("s")
        z16 = jnp.zeros((LANES,), jnp.float32)
        ones16 = jnp.ones((LANES,), jnp.float32)
        iota16 = jnp.arange(LANES, dtype=jnp.int32)

        @pl.loop(0, HROWS)
        def _(r):
            for j in range(128 // LANES):
                hist_v[r, pl.ds(j * LANES, LANES)] = z16

        @pl.when(sid == 0)
        def _():
            pltpu.sync_copy(hist_v, ssh)
            pltpu.sync_copy(hist_v, dsh)

        pltpu.sync_copy(iota_h, iid_v)
        plsc.subcore_barrier()

        base = sid * ept
        lo = half * cid
        hi = lo + half

        def grp(g, off):
            sv = sbuf[pl.ds(g * LANES, LANES)]
            dv = dbuf[pl.ds(g * LANES, LANES)]
            hv = jnp.where(cid == 0, sv, dv)
            plsc.addupdate_scatter(
                hist_v,
                [lax.shift_right_logical(hv, 7), lax.bitwise_and(hv, 127)],
                ones16,
            )
            keep = jnp.logical_and(dv >= lo, dv < hi)
            plsc.store_compressed(srcv.at[pl.ds(off, LANES)], sv, mask=keep)
            plsc.store_compressed(dstv.at[pl.ds(off, LANES)], dv - lo,
                                  mask=keep)
            cnt = lax.reduce_max(plsc.all_reduce_population_count(keep),
                                 axes=(0,))
            return off + cnt

        def chunk_body(ck, off):
            pltpu.sync_copy(src_h.at[pl.ds(base + ck * CHK, CHK)], sbuf)
            pltpu.sync_copy(dst_h.at[pl.ds(base + ck * CHK, CHK)], dbuf)
            return lax.fori_loop(0, CHK // LANES, grp, off)

        off = lax.fori_loop(0, ept // CHK, chunk_body, jnp.int32(0))

        for p in range(BLK * CHUNK // LANES):
            pad = iota16 + p * LANES
            srcv[pl.ds(off + p * LANES, LANES)] = lax.bitwise_and(pad, 1023)
            dstv[pl.ds(off + p * LANES, LANES)] = half + lax.bitwise_and(
                pad, DUMP - 1)

        nblk = lax.shift_right_logical(off + BLK * CHUNK - 1, 9)
        cntv[...] = jnp.broadcast_to(nblk, (LANES,)).astype(jnp.int32)

        @pl.when(cid == 0)
        def _():
            pltpu.sync_copy(hist_v, ssh.at[iid_v.at[0]], add=True)

        @pl.when(cid == 1)
        def _():
            pltpu.sync_copy(hist_v, dsh.at[iid_v.at[0]], add=True)

        pltpu.sync_copy(srcv, srcl_o.at[cid, sid])
        pltpu.sync_copy(dstv, dstl_o.at[cid, sid])
        pltpu.sync_copy(cntv, cnt_o.at[cid, sid])
        plsc.subcore_barrier()

        @pl.when(sid == 0)
        def _():
            pltpu.sync_copy(ssh, hist_o.at[cid, 0])
            pltpu.sync_copy(dsh, hist_o.at[cid, 1])

    return k(src_flat, dst_flat, iota_rows)


def _sc_scatter(h2, srcl, dstl4, cnts, half):
    n, d = h2.shape
    slot = srcl.shape[2]
    acc_rows = half + DUMP
    rows_per = 320
    rows_last = half - rows_per * (NS - 1)
    mesh = plsc.VectorSubcoreMesh(core_axis_name="c", subcore_axis_name="s")

    @functools.partial(
        pl.kernel,
        out_type=jax.ShapeDtypeStruct((NC, acc_rows, d), jnp.float32),
        mesh=mesh,
        scratch_types=[
            pltpu.VMEM((BLK * CHUNK,), jnp.int32),
            pltpu.VMEM((BLK, CHUNK), jnp.int32),
            pltpu.VMEM((2, CHUNK, d), jnp.float32),
            pltpu.VMEM((8, d), jnp.float32),
            pltpu.VMEM((16,), jnp.int32),
            pltpu.VMEM_SHARED((acc_rows, d), jnp.float32),
            pltpu.SemaphoreType.DMA,
            pltpu.SemaphoreType.DMA,
            pltpu.SemaphoreType.DMA,
            pltpu.SemaphoreType.DMA,
        ],
        compiler_params=_sc_compiler_params(),
    )
    def k(h2_h, sl_h, dl_h, cn_h, out_h, sblk, dblk, rows, zbuf, cntv,
          pooled, g0, g1, s0, s1):
        cid = lax.axis_index("c")
        sid = lax.axis_index("s")
        gsem = (g0, g1)
        ssem = (s0, s1)
        z16 = jnp.zeros((LANES,), jnp.float32)

        for i in range(8):
            for j in range(d // LANES):
                zbuf[i, pl.ds(j * LANES, LANES)] = z16

        r0 = sid * rows_per

        @pl.when(sid < NS - 1)
        def _():
            @pl.loop(0, rows_per // 8)
            def _(t):
                pltpu.sync_copy(zbuf, pooled.at[pl.ds(r0 + t * 8, 8)])

        @pl.when(sid == NS - 1)
        def _():
            @pl.loop(0, rows_last // 8)
            def _(t):
                pltpu.sync_copy(zbuf, pooled.at[pl.ds(r0 + t * 8, 8)])

        pltpu.sync_copy(cn_h.at[cid, sid], cntv)
        plsc.subcore_barrier()

        nblk = lax.reduce_max(cntv[...], axes=(0,))

        @pl.loop(0, nblk)
        def _(b):
            pltpu.sync_copy(sl_h.at[cid, sid, pl.ds(b * BLK * CHUNK,
                                                    BLK * CHUNK)], sblk)
            pltpu.sync_copy(dl_h.at[cid, sid, pl.ds(b * BLK, BLK)], dblk)
            pend_g = [None, None]
            pend_s = [None, None]
            for j in range(BLK):
                s = j & 1
                if pend_s[s] is not None:
                    pend_s[s].wait()
                pend_g[s] = pltpu.async_copy(
                    h2_h.at[sblk.at[pl.ds(j * CHUNK, CHUNK)]],
                    rows.at[s], gsem[s])
                if j >= 1:
                    o = s ^ 1
                    pend_g[o].wait()
                    pend_s[o] = pltpu.async_copy(
                        rows.at[o], pooled.at[dblk.at[j - 1]], ssem[o],
                        add=True)
            last = (BLK - 1) & 1
            pend_g[last].wait()
            pend_s[last] = pltpu.async_copy(
                rows.at[last], pooled.at[dblk.at[BLK - 1]], ssem[last],
                add=True)
            pend_s[last ^ 1].wait()
            pend_s[last].wait()

        plsc.subcore_barrier()

        @pl.when(sid < NS - 1)
        def _():
            pltpu.sync_copy(pooled.at[pl.ds(r0, rows_per)],
                            out_h.at[cid, pl.ds(r0, rows_per)])

        @pl.when(sid == NS - 1)
        def _():
            pltpu.sync_copy(pooled.at[pl.ds(r0, rows_last)],
                            out_h.at[cid, pl.ds(r0, rows_last)])

    return k(h2, srcl, dstl4, cnts)


def _dot(a, b):
    return lax.dot_general(a, b, (((1,), (0,)), ((), ())),
                           precision=lax.Precision.HIGHEST,
                           preferred_element_type=jnp.float32)


def _mm_scale_body(hist_ref, x_ref, w_ref, b_ref, h2_ref, rs_ref):
    out_deg = hist_ref[0, 0, :, :] + hist_ref[1, 0, :, :]
    in_deg = hist_ref[0, 1, :, :] + hist_ref[1, 1, :, :]
    rs_out = lax.rsqrt(jnp.maximum(out_deg, 1.0))
    h_relu = jnp.maximum(_dot(x_ref[...], w_ref[...]) + b_ref[...], 0.0)
    h2_ref[...] = h_relu * rs_out
    rs_ref[...] = lax.rsqrt(jnp.maximum(in_deg, 1.0))


def _tc_mm_scale(hist_n, x, w, b_row, block_rows):
    n, d = x.shape
    h = w.shape[1]
    grid = (n // block_rows,)
    return pl.pallas_call(
        _mm_scale_body,
        grid=grid,
        in_specs=[
            pl.BlockSpec((NC, 2, block_rows, 1), lambda i: (0, 0, i, 0)),
            pl.BlockSpec((block_rows, d), lambda i: (i, 0)),
            pl.BlockSpec((d, h), lambda i: (0, 0)),
            pl.BlockSpec((1, h), lambda i: (0, 0)),
        ],
        out_specs=[
            pl.BlockSpec((block_rows, h), lambda i: (i, 0)),
            pl.BlockSpec((block_rows, 1), lambda i: (i, 0)),
        ],
        out_shape=[
            jax.ShapeDtypeStruct((n, h), jnp.float32),
            jax.ShapeDtypeStruct((n, 1), jnp.float32),
        ],
    )(hist_n, x, w, b_row)


def _final_body(p_ref, rs_ref, w_ref, b_ref, x_ref, o_ref):
    s = p_ref[0] * rs_ref[...]
    o_ref[...] = jnp.maximum(_dot(s, w_ref[...]) + b_ref[...], 0.0) + x_ref[...]


def _tc_final(pooled2, rs_col, w, b_row, x, half):
    n, d = x.shape
    h = w.shape[1]
    block_rows = 1000
    per_half = half // block_rows
    grid = (n // block_rows,)
    return pl.pallas_call(
        _final_body,
        grid=grid,
        in_specs=[
            pl.BlockSpec((1, block_rows, h),
                         lambda i: (i // per_half, i % per_half, 0)),
            pl.BlockSpec((block_rows, 1), lambda i: (i, 0)),
            pl.BlockSpec((h, h), lambda i: (0, 0)),
            pl.BlockSpec((1, h), lambda i: (0, 0)),
            pl.BlockSpec((block_rows, d), lambda i: (i, 0)),
        ],
        out_specs=pl.BlockSpec((block_rows, h), lambda i: (i, 0)),
        out_shape=jax.ShapeDtypeStruct((n, h), jnp.float32),
    )(pooled2, rs_col, w, b_row, x)


def kernel(x, edge_index, W_edge, b_edge, W_node, b_node):
    x = x.astype(jnp.float32)
    ei = edge_index.astype(jnp.int32)
    src = ei[0]
    dst = ei[1]
    n = x.shape[0]
    half = n // 2
    iota_rows = jnp.arange(HROWS, dtype=jnp.int32).reshape(1, HROWS)

    hist, srcl, dstl, cnts = _sc_degrees_partition(src, dst, iota_rows, n)
    hist_n = hist.reshape(NC, 2, HROWS * 128)[:, :, :n, None]
    h2, rs_col = _tc_mm_scale(hist_n, x, W_edge, b_edge.reshape(1, -1), 2000)
    slot = srcl.shape[2]
    dstl4 = dstl.reshape(NC, NS, slot // CHUNK, CHUNK)
    pooled2 = _sc_scatter(h2, srcl, dstl4, cnts, half)
    out = _tc_final(pooled2, rs_col, W_node, b_node.reshape(1, -1), x, half)
    return out


# speedup vs baseline: 24.0630x; 1.0802x over previous
import dataclasses
import functools

import jax
import jax.numpy as jnp
from jax import lax
from jax.experimental import pallas as pl
from jax.experimental.pallas import tpu as pltpu
from jax.experimental.pallas import tpu_sc as plsc

NC = 2
NS = 16
NW = NC * NS
LANES = 16
HROWS = 80
CHUNK = 64
BLK = 8
DUMP = 64
CHK = 2000


def _sc_compiler_params():
    cp = pltpu.CompilerParams()
    if "needs_layout_passes" in pltpu.CompilerParams.__dataclass_fields__:
        cp = dataclasses.replace(cp, needs_layout_passes=False)
    return cp


def _sc_degrees_partition(src_flat, dst_flat, iota_rows, n):
    e = src_flat.shape[0]
    ept = e // NS
    assert ept * NS == e and ept % CHK == 0
    half = n // 2
    slot = (ept + 2 * BLK * CHUNK - 1) // (BLK * CHUNK) * (BLK * CHUNK)
    mesh = plsc.VectorSubcoreMesh(core_axis_name="c", subcore_axis_name="s")

    @functools.partial(
        pl.kernel,
        out_type=[
            jax.ShapeDtypeStruct((NC, 2, HROWS, 128), jnp.float32),
            jax.ShapeDtypeStruct((NC, NS, slot), jnp.int32),
            jax.ShapeDtypeStruct((NC, NS, slot), jnp.int32),
            jax.ShapeDtypeStruct((NC, NS, 16), jnp.int32),
        ],
        mesh=mesh,
        scratch_types=[
            pltpu.VMEM((CHK,), jnp.int32),
            pltpu.VMEM((CHK,), jnp.int32),
            pltpu.VMEM((HROWS, 128), jnp.float32),
            pltpu.VMEM((slot,), jnp.int32),
            pltpu.VMEM((slot,), jnp.int32),
            pltpu.VMEM((1, HROWS), jnp.int32),
            pltpu.VMEM((16,), jnp.int32),
            pltpu.VMEM_SHARED((HROWS, 128), jnp.float32),
            pltpu.VMEM_SHARED((HROWS, 128), jnp.float32),
        ],
        compiler_params=_sc_compiler_params(),
    )
    def k(src_h, dst_h, iota_h, hist_o, srcl_o, dstl_o, cnt_o,
          sbuf, dbuf, hist_v, srcv, dstv, iid_v, cntv, ssh, dsh):
        cid = lax.axis_index("c")
        sid = lax.axis_index("s")
        z16 = jnp.zeros((LANES,), jnp.float32)
        ones16 = jnp.ones((LANES,), jnp.float32)
        iota16 = jnp.arange(LANES, dtype=jnp.int32)

        @pl.loop(0, HROWS)
        def _(r):
            for j in range(128 // LANES):
                hist_v[r, pl.ds(j * LANES, LANES)] = z16

        @pl.when(sid == 0)
        def _():
            pltpu.sync_copy(hist_v, ssh)
            pltpu.sync_copy(hist_v, dsh)

        pltpu.sync_copy(iota_h, iid_v)
        plsc.subcore_barrier()

        @pl.loop(0, slot // LANES)
        def _(p):
            pad = iota16 + p * LANES
            srcv[pl.ds(p * LANES, LANES)] = lax.bitwise_and(pad, 1023)
            dstv[pl.ds(p * LANES, LANES)] = half + lax.bitwise_and(
                pad, DUMP - 1)

        base = sid * ept
        lo = half * cid
        hi = lo + half

        def grp(g, off):
            sv = sbuf[pl.ds(g * LANES, LANES)]
            dv = dbuf[pl.ds(g * LANES, LANES)]
            hv = jnp.where(cid == 0, sv, dv)
            plsc.addupdate_scatter(
                hist_v,
                [lax.shift_right_logical(hv, 7), lax.bitwise_and(hv, 127)],
                ones16,
            )
            keep = jnp.logical_and(dv >= lo, dv < hi)
            plsc.store_compressed(srcv.at[pl.ds(off, LANES)], sv, mask=keep)
            plsc.store_compressed(dstv.at[pl.ds(off, LANES)], dv - lo,
                                  mask=keep)
            cnt = lax.reduce_max(plsc.all_reduce_population_count(keep),
                                 axes=(0,))
            return off + cnt

        def chunk_body(ck, off):
            pltpu.sync_copy(src_h.at[pl.ds(base + ck * CHK, CHK)], sbuf)
            pltpu.sync_copy(dst_h.at[pl.ds(base + ck * CHK, CHK)], dbuf)
            return lax.fori_loop(0, CHK // LANES, grp, off)

        off = lax.fori_loop(0, ept // CHK, chunk_body, jnp.int32(0))

        nblk = lax.shift_right_logical(off + BLK * CHUNK - 1, 9)
        cntv[...] = jnp.broadcast_to(nblk, (LANES,)).astype(jnp.int32)

        @pl.when(cid == 0)
        def _():
            pltpu.sync_copy(hist_v, ssh.at[iid_v.at[0]], add=True)

        @pl.when(cid == 1)
        def _():
            pltpu.sync_copy(hist_v, dsh.at[iid_v.at[0]], add=True)

        pltpu.sync_copy(srcv, srcl_o.at[cid, sid])
        pltpu.sync_copy(dstv, dstl_o.at[cid, sid])
        pltpu.sync_copy(cntv, cnt_o.at[cid, sid])
        plsc.subcore_barrier()

        @pl.when(sid == 0)
        def _():
            pltpu.sync_copy(ssh, hist_o.at[cid, 0])
            pltpu.sync_copy(dsh, hist_o.at[cid, 1])

    return k(src_flat, dst_flat, iota_rows)


def _sc_scatter(h2, srcl, dstl4, cnts, half):
    n, d = h2.shape
    slot = srcl.shape[2]
    acc_rows = half + DUMP
    rows_per = 320
    rows_last = half - rows_per * (NS - 1)
    mesh = plsc.VectorSubcoreMesh(core_axis_name="c", subcore_axis_name="s")

    @functools.partial(
        pl.kernel,
        out_type=jax.ShapeDtypeStruct((NC, acc_rows, d), jnp.float32),
        mesh=mesh,
        scratch_types=[
            pltpu.VMEM((2, BLK * CHUNK), jnp.int32),
            pltpu.VMEM((2, BLK, CHUNK), jnp.int32),
            pltpu.VMEM((2, CHUNK, d), jnp.float32),
            pltpu.VMEM((8, d), jnp.float32),
            pltpu.VMEM((16,), jnp.int32),
            pltpu.VMEM_SHARED((acc_rows, d), jnp.float32),
            pltpu.SemaphoreType.DMA,
            pltpu.SemaphoreType.DMA,
            pltpu.SemaphoreType.DMA,
            pltpu.SemaphoreType.DMA,
            pltpu.SemaphoreType.DMA,
        ],
        compiler_params=_sc_compiler_params(),
    )
    def k(h2_h, sl_h, dl_h, cn_h, out_h, sblk, dblk, rows, zbuf, cntv,
          pooled, g0, g1, s0, s1, isem):
        cid = lax.axis_index("c")
        sid = lax.axis_index("s")
        gsem = (g0, g1)
        ssem = (s0, s1)
        z16 = jnp.zeros((LANES,), jnp.float32)

        for i in range(8):
            for j in range(d // LANES):
                zbuf[i, pl.ds(j * LANES, LANES)] = z16

        r0 = sid * rows_per

        @pl.when(sid < NS - 1)
        def _():
            @pl.loop(0, rows_per // 8)
            def _(t):
                pltpu.sync_copy(zbuf, pooled.at[pl.ds(r0 + t * 8, 8)])

        @pl.when(sid == NS - 1)
        def _():
            @pl.loop(0, rows_last // 8)
            def _(t):
                pltpu.sync_copy(zbuf, pooled.at[pl.ds(r0 + t * 8, 8)])

        pltpu.sync_copy(cn_h.at[cid, sid], cntv)
        plsc.subcore_barrier()

        nblk = lax.reduce_max(cntv[...], axes=(0,))

        TYP = 21
        pltpu.sync_copy(sl_h.at[cid, sid, pl.ds(0, BLK * CHUNK)], sblk.at[0])
        pltpu.sync_copy(dl_h.at[cid, sid, pl.ds(0, BLK)], dblk.at[0])
        pend_g = [None, None]
        pend_s = [None, None]
        pend_pi = None
        for b in range(TYP):
            ib = b & 1
            if pend_pi is not None:
                pend_pi[0].wait()
                pend_pi[1].wait()
                pend_pi = None
            for j in range(BLK):
                s = j & 1
                if pend_s[s] is not None:
                    pend_s[s].wait()
                pend_g[s] = pltpu.async_copy(
                    h2_h.at[sblk.at[ib, pl.ds(j * CHUNK, CHUNK)]],
                    rows.at[s], gsem[s])
                if b * BLK + j >= 1:
                    o = s ^ 1
                    jprev = j - 1 if j >= 1 else BLK - 1
                    ibprev = ib if j >= 1 else ib ^ 1
                    pend_g[o].wait()
                    pend_s[o] = pltpu.async_copy(
                        rows.at[o], pooled.at[dblk.at[ibprev, jprev]],
                        ssem[o], add=True)
                if j == 2 and b + 1 < TYP:
                    c1 = (b + 1) * BLK
                    pend_pi = (
                        pltpu.async_copy(
                            sl_h.at[cid, sid, pl.ds(c1 * CHUNK, BLK * CHUNK)],
                            sblk.at[ib ^ 1], isem),
                        pltpu.async_copy(
                            dl_h.at[cid, sid, pl.ds(c1, BLK)],
                            dblk.at[ib ^ 1], isem),
                    )
        last = (BLK - 1) & 1
        lb = (TYP - 1) & 1
        pend_g[last].wait()
        pend_s[last] = pltpu.async_copy(
            rows.at[last], pooled.at[dblk.at[lb, BLK - 1]], ssem[last],
            add=True)
        pend_s[last ^ 1].wait()
        pend_s[last].wait()

        @pl.loop(TYP, nblk)
        def _(b):
            pltpu.sync_copy(sl_h.at[cid, sid, pl.ds(b * BLK * CHUNK,
                                                    BLK * CHUNK)], sblk.at[0])
            pltpu.sync_copy(dl_h.at[cid, sid, pl.ds(b * BLK, BLK)],
                            dblk.at[0])
            pend_g = [None, None]
            pend_s = [None, None]
            for j in range(BLK):
                s = j & 1
                if pend_s[s] is not None:
                    pend_s[s].wait()
                pend_g[s] = pltpu.async_copy(
                    h2_h.at[sblk.at[0, pl.ds(j * CHUNK, CHUNK)]],
                    rows.at[s], gsem[s])
                if j >= 1:
                    o = s ^ 1
                    pend_g[o].wait()
                    pend_s[o] = pltpu.async_copy(
                        rows.at[o], pooled.at[dblk.at[0, j - 1]], ssem[o],
                        add=True)
            pend_g[last].wait()
            pend_s[last] = pltpu.async_copy(
                rows.at[last], pooled.at[dblk.at[0, BLK - 1]], ssem[last],
                add=True)
            pend_s[last ^ 1].wait()
            pend_s[last].wait()

        plsc.subcore_barrier()

        @pl.when(sid < NS - 1)
        def _():
            pltpu.sync_copy(pooled.at[pl.ds(r0, rows_per)],
                            out_h.at[cid, pl.ds(r0, rows_per)])

        @pl.when(sid == NS - 1)
        def _():
            pltpu.sync_copy(pooled.at[pl.ds(r0, rows_last)],
                            out_h.at[cid, pl.ds(r0, rows_last)])

    return k(h2, srcl, dstl4, cnts)


def _dot(a, b):
    return lax.dot_general(a, b, (((1,), (0,)), ((), ())),
                           precision=lax.Precision.HIGHEST,
                           preferred_element_type=jnp.float32)


def _mm_scale_body(hist_ref, x_ref, w_ref, b_ref, h2_ref, rs_ref):
    out_deg = hist_ref[0, 0, :, :] + hist_ref[1, 0, :, :]
    in_deg = hist_ref[0, 1, :, :] + hist_ref[1, 1, :, :]
    rs_out = lax.rsqrt(jnp.maximum(out_deg, 1.0))
    h_relu = jnp.maximum(_dot(x_ref[...], w_ref[...]) + b_ref[...], 0.0)
    h2_ref[...] = h_relu * rs_out
    rs_ref[...] = lax.rsqrt(jnp.maximum(in_deg, 1.0))


def _tc_mm_scale(hist_n, x, w, b_row, block_rows):
    n, d = x.shape
    h = w.shape[1]
    grid = (n // block_rows,)
    return pl.pallas_call(
        _mm_scale_body,
        grid=grid,
        in_specs=[
            pl.BlockSpec((NC, 2, block_rows, 1), lambda i: (0, 0, i, 0)),
            pl.BlockSpec((block_rows, d), lambda i: (i, 0)),
            pl.BlockSpec((d, h), lambda i: (0, 0)),
            pl.BlockSpec((1, h), lambda i: (0, 0)),
        ],
        out_specs=[
            pl.BlockSpec((block_rows, h), lambda i: (i, 0)),
            pl.BlockSpec((block_rows, 1), lambda i: (i, 0)),
        ],
        out_shape=[
            jax.ShapeDtypeStruct((n, h), jnp.float32),
            jax.ShapeDtypeStruct((n, 1), jnp.float32),
        ],
    )(hist_n, x, w, b_row)


def _final_body(p_ref, rs_ref, w_ref, b_ref, x_ref, o_ref):
    s = p_ref[0] * rs_ref[...]
    o_ref[...] = jnp.maximum(_dot(s, w_ref[...]) + b_ref[...], 0.0) + x_ref[...]


def _tc_final(pooled2, rs_col, w, b_row, x, half):
    n, d = x.shape
    h = w.shape[1]
    block_rows = 1000
    per_half = half // block_rows
    grid = (n // block_rows,)
    return pl.pallas_call(
        _final_body,
        grid=grid,
        in_specs=[
            pl.BlockSpec((1, block_rows, h),
                         lambda i: (i // per_half, i % per_half, 0)),
            pl.BlockSpec((block_rows, 1), lambda i: (i, 0)),
            pl.BlockSpec((h, h), lambda i: (0, 0)),
            pl.BlockSpec((1, h), lambda i: (0, 0)),
            pl.BlockSpec((block_rows, d), lambda i: (i, 0)),
        ],
        out_specs=pl.BlockSpec((block_rows, h), lambda i: (i, 0)),
        out_shape=jax.ShapeDtypeStruct((n, h), jnp.float32),
    )(pooled2, rs_col, w, b_row, x)


def kernel(x, edge_index, W_edge, b_edge, W_node, b_node):
    x = x.astype(jnp.float32)
    ei = edge_index.astype(jnp.int32)
    src = ei[0]
    dst = ei[1]
    n = x.shape[0]
    half = n // 2
    iota_rows = jnp.arange(HROWS, dtype=jnp.int32).reshape(1, HROWS)

    hist, srcl, dstl, cnts = _sc_degrees_partition(src, dst, iota_rows, n)
    hist_n = hist.reshape(NC, 2, HROWS * 128)[:, :, :n, None]
    h2, rs_col = _tc_mm_scale(hist_n, x, W_edge, b_edge.reshape(1, -1), 2000)
    slot = srcl.shape[2]
    dstl4 = dstl.reshape(NC, NS, slot // CHUNK, CHUNK)
    pooled2 = _sc_scatter(h2, srcl, dstl4, cnts, half)
    out = _tc_final(pooled2, rs_col, W_node, b_node.reshape(1, -1), x, half)
    return out


# speedup vs baseline: 24.2017x; 1.0058x over previous
import dataclasses
import functools

import jax
import jax.numpy as jnp
from jax import lax
from jax.experimental import pallas as pl
from jax.experimental.pallas import tpu as pltpu
from jax.experimental.pallas import tpu_sc as plsc

NC = 2
NS = 16
NW = NC * NS
LANES = 16
HROWS = 80
CHUNK = 64
BLK = 8
DUMP = 64
CHK = 2000


def _sc_compiler_params():
    cp = pltpu.CompilerParams()
    if "needs_layout_passes" in pltpu.CompilerParams.__dataclass_fields__:
        cp = dataclasses.replace(cp, needs_layout_passes=False)
    return cp


def _sc_degrees_partition(src_flat, dst_flat, iota_rows, n):
    e = src_flat.shape[0]
    ept = e // NS
    assert ept * NS == e and ept % CHK == 0
    half = n // 2
    slot = (ept + 2 * BLK * CHUNK - 1) // (BLK * CHUNK) * (BLK * CHUNK)
    mesh = plsc.VectorSubcoreMesh(core_axis_name="c", subcore_axis_name="s")

    @functools.partial(
        pl.kernel,
        out_type=[
            jax.ShapeDtypeStruct((NC, 2, HROWS, 128), jnp.float32),
            jax.ShapeDtypeStruct((NC, NS, slot), jnp.int32),
            jax.ShapeDtypeStruct((NC, NS, slot), jnp.int32),
            jax.ShapeDtypeStruct((NC, NS, 16), jnp.int32),
        ],
        mesh=mesh,
        scratch_types=[
            pltpu.VMEM((CHK,), jnp.int32),
            pltpu.VMEM((CHK,), jnp.int32),
            pltpu.VMEM((HROWS, 128), jnp.float32),
            pltpu.VMEM((slot,), jnp.int32),
            pltpu.VMEM((slot,), jnp.int32),
            pltpu.VMEM((1, HROWS), jnp.int32),
            pltpu.VMEM((16,), jnp.int32),
            pltpu.VMEM_SHARED((HROWS, 128), jnp.float32),
            pltpu.VMEM_SHARED((HROWS, 128), jnp.float32),
        ],
        compiler_params=_sc_compiler_params(),
    )
    def k(src_h, dst_h, iota_h, hist_o, srcl_o, dstl_o, cnt_o,
          sbuf, dbuf, hist_v, srcv, dstv, iid_v, cntv, ssh, dsh):
        cid = lax.axis_index("c")
        sid = lax.axis_index("s")
        z16 = jnp.zeros((LANES,), jnp.float32)
        ones16 = jnp.ones((LANES,), jnp.float32)
        iota16 = jnp.arange(LANES, dtype=jnp.int32)

        @pl.loop(0, HROWS)
        def _(r):
            for j in range(128 // LANES):
                hist_v[r, pl.ds(j * LANES, LANES)] = z16

        @pl.when(sid == 0)
        def _():
            pltpu.sync_copy(hist_v, ssh)
            pltpu.sync_copy(hist_v, dsh)

        pltpu.sync_copy(iota_h, iid_v)
        plsc.subcore_barrier()

        @pl.loop(0, slot // LANES)
        def _(p):
            pad = iota16 + p * LANES
            srcv[pl.ds(p * LANES, LANES)] = lax.bitwise_and(pad, 1023)
            dstv[pl.ds(p * LANES, LANES)] = half + lax.bitwise_and(
                pad, DUMP - 1)

        base = sid * ept
        lo = half * cid
        hi = lo + half

        def grp(g, off):
            sv = sbuf[pl.ds(g * LANES, LANES)]
            dv = dbuf[pl.ds(g * LANES, LANES)]
            hv = jnp.where(cid == 0, sv, dv)
            plsc.addupdate_scatter(
                hist_v,
                [lax.shift_right_logical(hv, 7), lax.bitwise_and(hv, 127)],
                ones16,
            )
            keep = jnp.logical_and(dv >= lo, dv < hi)
            plsc.store_compressed(srcv.at[pl.ds(off, LANES)], sv, mask=keep)
            plsc.store_compressed(dstv.at[pl.ds(off, LANES)], dv - lo,
                                  mask=keep)
            cnt = lax.reduce_max(plsc.all_reduce_population_count(keep),
                                 axes=(0,))
            return off + cnt

        def chunk_body(ck, off):
            pltpu.sync_copy(src_h.at[pl.ds(base + ck * CHK, CHK)], sbuf)
            pltpu.sync_copy(dst_h.at[pl.ds(base + ck * CHK, CHK)], dbuf)
            return lax.fori_loop(0, CHK // LANES, grp, off)

        off = lax.fori_loop(0, ept // CHK, chunk_body, jnp.int32(0))

        nblk = lax.shift_right_logical(off + BLK * CHUNK - 1, 9)
        cntv[...] = jnp.broadcast_to(nblk, (LANES,)).astype(jnp.int32)

        @pl.when(cid == 0)
        def _():
            pltpu.sync_copy(hist_v, ssh.at[iid_v.at[0]], add=True)

        @pl.when(cid == 1)
        def _():
            pltpu.sync_copy(hist_v, dsh.at[iid_v.at[0]], add=True)

        pltpu.sync_copy(srcv, srcl_o.at[cid, sid])
        pltpu.sync_copy(dstv, dstl_o.at[cid, sid])
        pltpu.sync_copy(cntv, cnt_o.at[cid, sid])
        plsc.subcore_barrier()

        @pl.when(sid == 0)
        def _():
            pltpu.sync_copy(ssh, hist_o.at[cid, 0])
            pltpu.sync_copy(dsh, hist_o.at[cid, 1])

    return k(src_flat, dst_flat, iota_rows)


def _sc_scatter(h2, srcl, dstl4, cnts, half):
    n, d = h2.shape
    slot = srcl.shape[2]
    acc_rows = half + DUMP
    rows_per = 320
    rows_last = half - rows_per * (NS - 1)
    mesh = plsc.VectorSubcoreMesh(core_axis_name="c", subcore_axis_name="s")

    @functools.partial(
        pl.kernel,
        out_type=jax.ShapeDtypeStruct((NC, acc_rows, d), jnp.float32),
        mesh=mesh,
        scratch_types=[
            pltpu.VMEM((2, BLK * CHUNK), jnp.int32),
            pltpu.VMEM((2, BLK, CHUNK), jnp.int32),
            pltpu.VMEM((2, CHUNK, d), jnp.float32),
            pltpu.VMEM((40, d), jnp.float32),
            pltpu.VMEM((16,), jnp.int32),
            pltpu.VMEM_SHARED((acc_rows, d), jnp.float32),
            pltpu.SemaphoreType.DMA,
            pltpu.SemaphoreType.DMA,
            pltpu.SemaphoreType.DMA,
            pltpu.SemaphoreType.DMA,
            pltpu.SemaphoreType.DMA,
        ],
        compiler_params=_sc_compiler_params(),
    )
    def k(h2_h, sl_h, dl_h, cn_h, out_h, sblk, dblk, rows, zbuf, cntv,
          pooled, g0, g1, s0, s1, isem):
        cid = lax.axis_index("c")
        sid = lax.axis_index("s")
        gsem = (g0, g1)
        ssem = (s0, s1)
        z16 = jnp.zeros((LANES,), jnp.float32)

        for i in range(40):
            for j in range(d // LANES):
                zbuf[i, pl.ds(j * LANES, LANES)] = z16

        r0 = sid * rows_per

        @pl.when(sid < NS - 1)
        def _():
            zd = [pltpu.async_copy(zbuf, pooled.at[pl.ds(r0 + t * 40, 40)],
                                   isem)
                  for t in range(rows_per // 40)]
            for dsc in zd:
                dsc.wait()

        @pl.when(sid == NS - 1)
        def _():
            zd = [pltpu.async_copy(zbuf, pooled.at[pl.ds(r0 + t * 40, 40)],
                                   isem)
                  for t in range(rows_last // 40)]
            for dsc in zd:
                dsc.wait()

        pltpu.sync_copy(cn_h.at[cid, sid], cntv)
        plsc.subcore_barrier()

        nblk = lax.reduce_max(cntv[...], axes=(0,))

        TYP = 21
        pltpu.sync_copy(sl_h.at[cid, sid, pl.ds(0, BLK * CHUNK)], sblk.at[0])
        pltpu.sync_copy(dl_h.at[cid, sid, pl.ds(0, BLK)], dblk.at[0])
        pend_g = [None, None]
        pend_s = [None, None]
        pend_pi = None
        for b in range(TYP):
            ib = b & 1
            if pend_pi is not None:
                pend_pi[0].wait()
                pend_pi[1].wait()
                pend_pi = None
            for j in range(BLK):
                s = j & 1
                if pend_s[s] is not None:
                    pend_s[s].wait()
                pend_g[s] = pltpu.async_copy(
                    h2_h.at[sblk.at[ib, pl.ds(j * CHUNK, CHUNK)]],
                    rows.at[s], gsem[s])
                if b * BLK + j >= 1:
                    o = s ^ 1
                    jprev = j - 1 if j >= 1 else BLK - 1
                    ibprev = ib if j >= 1 else ib ^ 1
                    pend_g[o].wait()
                    pend_s[o] = pltpu.async_copy(
                        rows.at[o], pooled.at[dblk.at[ibprev, jprev]],
                        ssem[o], add=True)
                if j == 2 and b + 1 < TYP:
                    c1 = (b + 1) * BLK
                    pend_pi = (
                        pltpu.async_copy(
                            sl_h.at[cid, sid, pl.ds(c1 * CHUNK, BLK * CHUNK)],
                            sblk.at[ib ^ 1], isem),
                        pltpu.async_copy(
                            dl_h.at[cid, sid, pl.ds(c1, BLK)],
                            dblk.at[ib ^ 1], isem),
                    )
        last = (BLK - 1) & 1
        lb = (TYP - 1) & 1
        pend_g[last].wait()
        pend_s[last] = pltpu.async_copy(
            rows.at[last], pooled.at[dblk.at[lb, BLK - 1]], ssem[last],
            add=True)
        pend_s[last ^ 1].wait()
        pend_s[last].wait()

        @pl.loop(TYP, nblk)
        def _(b):
            pltpu.sync_copy(sl_h.at[cid, sid, pl.ds(b * BLK * CHUNK,
                                                    BLK * CHUNK)], sblk.at[0])
            pltpu.sync_copy(dl_h.at[cid, sid, pl.ds(b * BLK, BLK)],
                            dblk.at[0])
            pend_g = [None, None]
            pend_s = [None, None]
            for j in range(BLK):
                s = j & 1
                if pend_s[s] is not None:
                    pend_s[s].wait()
                pend_g[s] = pltpu.async_copy(
                    h2_h.at[sblk.at[0, pl.ds(j * CHUNK, CHUNK)]],
                    rows.at[s], gsem[s])
                if j >= 1:
                    o = s ^ 1
                    pend_g[o].wait()
                    pend_s[o] = pltpu.async_copy(
                        rows.at[o], pooled.at[dblk.at[0, j - 1]], ssem[o],
                        add=True)
            pend_g[last].wait()
            pend_s[last] = pltpu.async_copy(
                rows.at[last], pooled.at[dblk.at[0, BLK - 1]], ssem[last],
                add=True)
            pend_s[last ^ 1].wait()
            pend_s[last].wait()

        plsc.subcore_barrier()

        @pl.when(sid < NS - 1)
        def _():
            pltpu.sync_copy(pooled.at[pl.ds(r0, rows_per)],
                            out_h.at[cid, pl.ds(r0, rows_per)])

        @pl.when(sid == NS - 1)
        def _():
            pltpu.sync_copy(pooled.at[pl.ds(r0, rows_last)],
                            out_h.at[cid, pl.ds(r0, rows_last)])

    return k(h2, srcl, dstl4, cnts)


def _dot(a, b):
    return lax.dot_general(a, b, (((1,), (0,)), ((), ())),
                           precision=lax.Precision.HIGHEST,
                           preferred_element_type=jnp.float32)


def _mm_scale_body(hist_ref, x_ref, w_ref, b_ref, h2_ref, rs_ref):
    out_deg = hist_ref[0, 0, :, :] + hist_ref[1, 0, :, :]
    in_deg = hist_ref[0, 1, :, :] + hist_ref[1, 1, :, :]
    rs_out = lax.rsqrt(jnp.maximum(out_deg, 1.0))
    h_relu = jnp.maximum(_dot(x_ref[...], w_ref[...]) + b_ref[...], 0.0)
    h2_ref[...] = h_relu * rs_out
    rs_ref[...] = lax.rsqrt(jnp.maximum(in_deg, 1.0))


def _tc_mm_scale(hist_n, x, w, b_row, block_rows):
    n, d = x.shape
    h = w.shape[1]
    grid = (n // block_rows,)
    return pl.pallas_call(
        _mm_scale_body,
        grid=grid,
        in_specs=[
            pl.BlockSpec((NC, 2, block_rows, 1), lambda i: (0, 0, i, 0)),
            pl.BlockSpec((block_rows, d), lambda i: (i, 0)),
            pl.BlockSpec((d, h), lambda i: (0, 0)),
            pl.BlockSpec((1, h), lambda i: (0, 0)),
        ],
        out_specs=[
            pl.BlockSpec((block_rows, h), lambda i: (i, 0)),
            pl.BlockSpec((block_rows, 1), lambda i: (i, 0)),
        ],
        out_shape=[
            jax.ShapeDtypeStruct((n, h), jnp.float32),
            jax.ShapeDtypeStruct((n, 1), jnp.float32),
        ],
    )(hist_n, x, w, b_row)


def _final_body(p_ref, rs_ref, w_ref, b_ref, x_ref, o_ref):
    s = p_ref[0] * rs_ref[...]
    o_ref[...] = jnp.maximum(_dot(s, w_ref[...]) + b_ref[...], 0.0) + x_ref[...]


def _tc_final(pooled2, rs_col, w, b_row, x, half):
    n, d = x.shape
    h = w.shape[1]
    block_rows = 1000
    per_half = half // block_rows
    grid = (n // block_rows,)
    return pl.pallas_call(
        _final_body,
        grid=grid,
        in_specs=[
            pl.BlockSpec((1, block_rows, h),
                         lambda i: (i // per_half, i % per_half, 0)),
            pl.BlockSpec((block_rows, 1), lambda i: (i, 0)),
            pl.BlockSpec((h, h), lambda i: (0, 0)),
            pl.BlockSpec((1, h), lambda i: (0, 0)),
            pl.BlockSpec((block_rows, d), lambda i: (i, 0)),
        ],
        out_specs=pl.BlockSpec((block_rows, h), lambda i: (i, 0)),
        out_shape=jax.ShapeDtypeStruct((n, h), jnp.float32),
    )(pooled2, rs_col, w, b_row, x)


def kernel(x, edge_index, W_edge, b_edge, W_node, b_node):
    x = x.astype(jnp.float32)
    ei = edge_index.astype(jnp.int32)
    src = ei[0]
    dst = ei[1]
    n = x.shape[0]
    half = n // 2
    iota_rows = jnp.arange(HROWS, dtype=jnp.int32).reshape(1, HROWS)

    hist, srcl, dstl, cnts = _sc_degrees_partition(src, dst, iota_rows, n)
    hist_n = hist.reshape(NC, 2, HROWS * 128)[:, :, :n, None]
    h2, rs_col = _tc_mm_scale(hist_n, x, W_edge, b_edge.reshape(1, -1), 2000)
    slot = srcl.shape[2]
    dstl4 = dstl.reshape(NC, NS, slot // CHUNK, CHUNK)
    pooled2 = _sc_scatter(h2, srcl, dstl4, cnts, half)
    out = _tc_final(pooled2, rs_col, W_node, b_node.reshape(1, -1), x, half)
    return out


# speedup vs baseline: 24.4133x; 1.0087x over previous
import dataclasses
import functools

import jax
import jax.numpy as jnp
from jax import lax
from jax.experimental import pallas as pl
from jax.experimental.pallas import tpu as pltpu
from jax.experimental.pallas import tpu_sc as plsc

NC = 2
NS = 16
NW = NC * NS
LANES = 16
HROWS = 80
CHUNK = 64
BLK = 8
DUMP = 64
CHK = 2000


def _sc_compiler_params():
    cp = pltpu.CompilerParams()
    if "needs_layout_passes" in pltpu.CompilerParams.__dataclass_fields__:
        cp = dataclasses.replace(cp, needs_layout_passes=False)
    return cp


def _sc_degrees_partition(src_flat, dst_flat, iota_rows, n):
    e = src_flat.shape[0]
    ept = e // NS
    assert ept * NS == e and ept % CHK == 0
    half = n // 2
    slot = (ept + 2 * BLK * CHUNK - 1) // (BLK * CHUNK) * (BLK * CHUNK)
    mesh = plsc.VectorSubcoreMesh(core_axis_name="c", subcore_axis_name="s")

    @functools.partial(
        pl.kernel,
        out_type=[
            jax.ShapeDtypeStruct((NC, 2, HROWS, 128), jnp.float32),
            jax.ShapeDtypeStruct((NC, NS, slot), jnp.int32),
            jax.ShapeDtypeStruct((NC, NS, slot), jnp.int32),
            jax.ShapeDtypeStruct((NC, NS, 16), jnp.int32),
        ],
        mesh=mesh,
        scratch_types=[
            pltpu.VMEM((CHK,), jnp.int32),
            pltpu.VMEM((CHK,), jnp.int32),
            pltpu.VMEM((HROWS, 128), jnp.float32),
            pltpu.VMEM((slot,), jnp.int32),
            pltpu.VMEM((slot,), jnp.int32),
            pltpu.VMEM((1, HROWS), jnp.int32),
            pltpu.VMEM((16,), jnp.int32),
            pltpu.VMEM_SHARED((HROWS, 128), jnp.float32),
            pltpu.VMEM_SHARED((HROWS, 128), jnp.float32),
        ],
        compiler_params=_sc_compiler_params(),
    )
    def k(src_h, dst_h, iota_h, hist_o, srcl_o, dstl_o, cnt_o,
          sbuf, dbuf, hist_v, srcv, dstv, iid_v, cntv, ssh, dsh):
        cid = lax.axis_index("c")
        sid = lax.axis_index("s")
        z16 = jnp.zeros((LANES,), jnp.float32)
        ones16 = jnp.ones((LANES,), jnp.float32)
        iota16 = jnp.arange(LANES, dtype=jnp.int32)

        @pl.loop(0, HROWS)
        def _(r):
            for j in range(128 // LANES):
                hist_v[r, pl.ds(j * LANES, LANES)] = z16

        @pl.when(sid == 0)
        def _():
            pltpu.sync_copy(hist_v, ssh)
            pltpu.sync_copy(hist_v, dsh)

        pltpu.sync_copy(iota_h, iid_v)
        plsc.subcore_barrier()

        @pl.loop(0, 21 * BLK * CHUNK // LANES)
        def _(p):
            pad = iota16 + p * LANES
            srcv[pl.ds(p * LANES, LANES)] = lax.bitwise_and(pad, 1023)
            dstv[pl.ds(p * LANES, LANES)] = half + lax.bitwise_and(
                pad, DUMP - 1)

        base = sid * ept
        lo = half * cid
        hi = lo + half

        def grp(g, off):
            sv = sbuf[pl.ds(g * LANES, LANES)]
            dv = dbuf[pl.ds(g * LANES, LANES)]
            hv = jnp.where(cid == 0, sv, dv)
            plsc.addupdate_scatter(
                hist_v,
                [lax.shift_right_logical(hv, 7), lax.bitwise_and(hv, 127)],
                ones16,
            )
            keep = jnp.logical_and(dv >= lo, dv < hi)
            plsc.store_compressed(srcv.at[pl.ds(off, LANES)], sv, mask=keep)
            plsc.store_compressed(dstv.at[pl.ds(off, LANES)], dv - lo,
                                  mask=keep)
            cnt = lax.reduce_max(plsc.all_reduce_population_count(keep),
                                 axes=(0,))
            return off + cnt

        def chunk_body(ck, off):
            pltpu.sync_copy(src_h.at[pl.ds(base + ck * CHK, CHK)], sbuf)
            pltpu.sync_copy(dst_h.at[pl.ds(base + ck * CHK, CHK)], dbuf)
            return lax.fori_loop(0, CHK // LANES, grp, off)

        off = lax.fori_loop(0, ept // CHK, chunk_body, jnp.int32(0))

        for p in range(BLK * CHUNK // LANES):
            pad = iota16 + p * LANES
            srcv[pl.ds(off + p * LANES, LANES)] = lax.bitwise_and(pad, 1023)
            dstv[pl.ds(off + p * LANES, LANES)] = half + lax.bitwise_and(
                pad, DUMP - 1)

        nblk = lax.shift_right_logical(off + BLK * CHUNK - 1, 9)
        cntv[...] = jnp.broadcast_to(nblk, (LANES,)).astype(jnp.int32)

        @pl.when(cid == 0)
        def _():
            pltpu.sync_copy(hist_v, ssh.at[iid_v.at[0]], add=True)

        @pl.when(cid == 1)
        def _():
            pltpu.sync_copy(hist_v, dsh.at[iid_v.at[0]], add=True)

        pltpu.sync_copy(srcv, srcl_o.at[cid, sid])
        pltpu.sync_copy(dstv, dstl_o.at[cid, sid])
        pltpu.sync_copy(cntv, cnt_o.at[cid, sid])
        plsc.subcore_barrier()

        @pl.when(sid == 0)
        def _():
            pltpu.sync_copy(ssh, hist_o.at[cid, 0])
            pltpu.sync_copy(dsh, hist_o.at[cid, 1])

    return k(src_flat, dst_flat, iota_rows)


def _sc_scatter(h2, srcl, dstl4, cnts, half):
    n, d = h2.shape
    slot = srcl.shape[2]
    acc_rows = half + DUMP
    rows_per = 320
    rows_last = half - rows_per * (NS - 1)
    mesh = plsc.VectorSubcoreMesh(core_axis_name="c", subcore_axis_name="s")

    @functools.partial(
        pl.kernel,
        out_type=jax.ShapeDtypeStruct((NC, acc_rows, d), jnp.float32),
        mesh=mesh,
        scratch_types=[
            pltpu.VMEM((2, BLK * CHUNK), jnp.int32),
            pltpu.VMEM((2, BLK, CHUNK), jnp.int32),
            pltpu.VMEM((2, CHUNK, d), jnp.float32),
            pltpu.VMEM((40, d), jnp.float32),
            pltpu.VMEM((16,), jnp.int32),
            pltpu.VMEM_SHARED((acc_rows, d), jnp.float32),
            pltpu.SemaphoreType.DMA,
            pltpu.SemaphoreType.DMA,
            pltpu.SemaphoreType.DMA,
            pltpu.SemaphoreType.DMA,
            pltpu.SemaphoreType.DMA,
        ],
        compiler_params=_sc_compiler_params(),
    )
    def k(h2_h, sl_h, dl_h, cn_h, out_h, sblk, dblk, rows, zbuf, cntv,
          pooled, g0, g1, s0, s1, isem):
        cid = lax.axis_index("c")
        sid = lax.axis_index("s")
        gsem = (g0, g1)
        ssem = (s0, s1)
        z16 = jnp.zeros((LANES,), jnp.float32)

        for i in range(40):
            for j in range(d // LANES):
                zbuf[i, pl.ds(j * LANES, LANES)] = z16

        r0 = sid * rows_per

        @pl.when(sid < NS - 1)
        def _():
            zd = [pltpu.async_copy(zbuf, pooled.at[pl.ds(r0 + t * 40, 40)],
                                   isem)
                  for t in range(rows_per // 40)]
            for dsc in zd:
                dsc.wait()

        @pl.when(sid == NS - 1)
        def _():
            zd = [pltpu.async_copy(zbuf, pooled.at[pl.ds(r0 + t * 40, 40)],
                                   isem)
                  for t in range(rows_last // 40)]
            for dsc in zd:
                dsc.wait()

        pltpu.sync_copy(cn_h.at[cid, sid], cntv)
        plsc.subcore_barrier()

        nblk = lax.reduce_max(cntv[...], axes=(0,))

        TYP = 21
        pltpu.sync_copy(sl_h.at[cid, sid, pl.ds(0, BLK * CHUNK)], sblk.at[0])
        pltpu.sync_copy(dl_h.at[cid, sid, pl.ds(0, BLK)], dblk.at[0])
        pend_g = [None, None]
        pend_s = [None, None]
        pend_pi = None
        for b in range(TYP):
            ib = b & 1
            if pend_pi is not None:
                pend_pi[0].wait()
                pend_pi[1].wait()
                pend_pi = None
            for j in range(BLK):
                s = j & 1
                if pend_s[s] is not None:
                    pend_s[s].wait()
                pend_g[s] = pltpu.async_copy(
                    h2_h.at[sblk.at[ib, pl.ds(j * CHUNK, CHUNK)]],
                    rows.at[s], gsem[s])
                if b * BLK + j >= 1:
                    o = s ^ 1
                    jprev = j - 1 if j >= 1 else BLK - 1
                    ibprev = ib if j >= 1 else ib ^ 1
                    pend_g[o].wait()
                    pend_s[o] = pltpu.async_copy(
                        rows.at[o], pooled.at[dblk.at[ibprev, jprev]],
                        ssem[o], add=True)
                if j == 2 and b + 1 < TYP:
                    c1 = (b + 1) * BLK
                    pend_pi = (
                        pltpu.async_copy(
                            sl_h.at[cid, sid, pl.ds(c1 * CHUNK, BLK * CHUNK)],
                            sblk.at[ib ^ 1], isem),
                        pltpu.async_copy(
                            dl_h.at[cid, sid, pl.ds(c1, BLK)],
                            dblk.at[ib ^ 1], isem),
                    )
        last = (BLK - 1) & 1
        lb = (TYP - 1) & 1
        pend_g[last].wait()
        pend_s[last] = pltpu.async_copy(
            rows.at[last], pooled.at[dblk.at[lb, BLK - 1]], ssem[last],
            add=True)
        pend_s[last ^ 1].wait()
        pend_s[last].wait()

        @pl.loop(TYP, nblk)
        def _(b):
            pltpu.sync_copy(sl_h.at[cid, sid, pl.ds(b * BLK * CHUNK,
                                                    BLK * CHUNK)], sblk.at[0])
            pltpu.sync_copy(dl_h.at[cid, sid, pl.ds(b * BLK, BLK)],
                            dblk.at[0])
            pend_g = [None, None]
            pend_s = [None, None]
            for j in range(BLK):
                s = j & 1
                if pend_s[s] is not None:
                    pend_s[s].wait()
                pend_g[s] = pltpu.async_copy(
                    h2_h.at[sblk.at[0, pl.ds(j * CHUNK, CHUNK)]],
                    rows.at[s], gsem[s])
                if j >= 1:
                    o = s ^ 1
                    pend_g[o].wait()
                    pend_s[o] = pltpu.async_copy(
                        rows.at[o], pooled.at[dblk.at[0, j - 1]], ssem[o],
                        add=True)
            pend_g[last].wait()
            pend_s[last] = pltpu.async_copy(
                rows.at[last], pooled.at[dblk.at[0, BLK - 1]], ssem[last],
                add=True)
            pend_s[last ^ 1].wait()
            pend_s[last].wait()

        plsc.subcore_barrier()

        @pl.when(sid < NS - 1)
        def _():
            pltpu.sync_copy(pooled.at[pl.ds(r0, rows_per)],
                            out_h.at[cid, pl.ds(r0, rows_per)])

        @pl.when(sid == NS - 1)
        def _():
            pltpu.sync_copy(pooled.at[pl.ds(r0, rows_last)],
                            out_h.at[cid, pl.ds(r0, rows_last)])

    return k(h2, srcl, dstl4, cnts)


def _dot(a, b):
    return lax.dot_general(a, b, (((1,), (0,)), ((), ())),
                           precision=lax.Precision.HIGHEST,
                           preferred_element_type=jnp.float32)


def _mm_scale_body(hist_ref, x_ref, w_ref, b_ref, h2_ref, rs_ref):
    out_deg = hist_ref[0, 0, :, :] + hist_ref[1, 0, :, :]
    in_deg = hist_ref[0, 1, :, :] + hist_ref[1, 1, :, :]
    rs_out = lax.rsqrt(jnp.maximum(out_deg, 1.0))
    h_relu = jnp.maximum(_dot(x_ref[...], w_ref[...]) + b_ref[...], 0.0)
    h2_ref[...] = h_relu * rs_out
    rs_ref[...] = lax.rsqrt(jnp.maximum(in_deg, 1.0))


def _tc_mm_scale(hist_n, x, w, b_row, block_rows):
    n, d = x.shape
    h = w.shape[1]
    grid = (n // block_rows,)
    return pl.pallas_call(
        _mm_scale_body,
        grid=grid,
        in_specs=[
            pl.BlockSpec((NC, 2, block_rows, 1), lambda i: (0, 0, i, 0)),
            pl.BlockSpec((block_rows, d), lambda i: (i, 0)),
            pl.BlockSpec((d, h), lambda i: (0, 0)),
            pl.BlockSpec((1, h), lambda i: (0, 0)),
        ],
        out_specs=[
            pl.BlockSpec((block_rows, h), lambda i: (i, 0)),
            pl.BlockSpec((block_rows, 1), lambda i: (i, 0)),
        ],
        out_shape=[
            jax.ShapeDtypeStruct((n, h), jnp.float32),
            jax.ShapeDtypeStruct((n, 1), jnp.float32),
        ],
    )(hist_n, x, w, b_row)


def _final_body(p_ref, rs_ref, w_ref, b_ref, x_ref, o_ref):
    s = p_ref[0] * rs_ref[...]
    o_ref[...] = jnp.maximum(_dot(s, w_ref[...]) + b_ref[...], 0.0) + x_ref[...]


def _tc_final(pooled2, rs_col, w, b_row, x, half):
    n, d = x.shape
    h = w.shape[1]
    block_rows = 1000
    per_half = half // block_rows
    grid = (n // block_rows,)
    return pl.pallas_call(
        _final_body,
        grid=grid,
        in_specs=[
            pl.BlockSpec((1, block_rows, h),
                         lambda i: (i // per_half, i % per_half, 0)),
            pl.BlockSpec((block_rows, 1), lambda i: (i, 0)),
            pl.BlockSpec((h, h), lambda i: (0, 0)),
            pl.BlockSpec((1, h), lambda i: (0, 0)),
            pl.BlockSpec((block_rows, d), lambda i: (i, 0)),
        ],
        out_specs=pl.BlockSpec((block_rows, h), lambda i: (i, 0)),
        out_shape=jax.ShapeDtypeStruct((n, h), jnp.float32),
    )(pooled2, rs_col, w, b_row, x)


def kernel(x, edge_index, W_edge, b_edge, W_node, b_node):
    x = x.astype(jnp.float32)
    ei = edge_index.astype(jnp.int32)
    src = ei[0]
    dst = ei[1]
    n = x.shape[0]
    half = n // 2
    iota_rows = jnp.arange(HROWS, dtype=jnp.int32).reshape(1, HROWS)

    hist, srcl, dstl, cnts = _sc_degrees_partition(src, dst, iota_rows, n)
    hist_n = hist.reshape(NC, 2, HROWS * 128)[:, :, :n, None]
    h2, rs_col = _tc_mm_scale(hist_n, x, W_edge, b_edge.reshape(1, -1), 2000)
    slot = srcl.shape[2]
    dstl4 = dstl.reshape(NC, NS, slot // CHUNK, CHUNK)
    pooled2 = _sc_scatter(h2, srcl, dstl4, cnts, half)
    out = _tc_final(pooled2, rs_col, W_node, b_node.reshape(1, -1), x, half)
    return out
